# baseline (speedup 1.0000x reference)
import functools

import jax
import jax.numpy as jnp
from jax import lax
from jax.experimental import pallas as pl
from jax.experimental.pallas import tpu as pltpu

F32 = jnp.float32
_N = 1024
_E = 131072
_B = 32
_IND = 128
_EDD = 32
_LGNN = 5
_NAF = 16
_LAT = 64
ET = 2048
_HIGH = jax.lax.Precision.DEFAULT


def _dot(a, b):
    return jnp.dot(a, b, precision=_HIGH, preferred_element_type=F32)


def _silu(v):
    return v * jax.nn.sigmoid(v)


def _onehot(idx_col, width):
    t = idx_col.shape[0]
    cols = lax.broadcasted_iota(jnp.int32, (t, width), 1)
    return jnp.where(idx_col == cols, 1.0, 0.0).astype(F32)


def _node_init_body(x_ref, z_ref, b_ref, t_ref, tmaW_ref, tmab_ref,
                    amW_ref, amb_ref, atmW_ref, atmb_ref, lmW_ref, lmb_ref,
                    s_ref):
    ta = _dot(t_ref[...], tmaW_ref[...]) + tmab_ref[...]
    t2 = _dot(ta, atmW_ref[...])
    a1 = _dot(amW_ref[...], atmW_ref[...])
    c2 = _dot(amb_ref[...], atmW_ref[...]) + atmb_ref[...] + lmb_ref[...]
    oh = _onehot(b_ref[...], _B)
    s_ref[...] = (_dot(x_ref[...], a1) + _dot(oh, t2)
                  + _dot(z_ref[...], lmW_ref[...]) + c2)


def _node_init(x, z, batch_col, t, p):
    return pl.pallas_call(
        _node_init_body,
        out_shape=jax.ShapeDtypeStruct((_N, _IND), F32),
    )(x, z, batch_col, t, p['tma_W'], p['tma_b'].reshape(1, -1),
      p['am_W'], p['am_b'].reshape(1, -1), p['atm_W'],
      p['atm_b'].reshape(1, -1), p['lm_W'], p['lm_b'].reshape(1, -1))


def _edge_init_body(ea_ref, beg_ref, t_ref, tmbW_ref, tmbb_ref,
                    bmW_ref, bmb_ref, btmW_ref, btmb_ref, e_ref):
    tb = _dot(t_ref[...], tmbW_ref[...]) + tmbb_ref[...]
    t1 = _dot(tb, btmW_ref[...])
    w1 = _dot(bmW_ref[...], btmW_ref[...])
    c1 = _dot(bmb_ref[...], btmW_ref[...]) + btmb_ref[...]
    oh = _onehot(beg_ref[...], _B)
    e_ref[...] = _dot(ea_ref[...], w1) + _dot(oh, t1) + c1


def _edge_init(edge_attr, beg_col, t, p):
    nbt = edge_attr.shape[1]
    grid = (_E // ET,)
    return pl.pallas_call(
        _edge_init_body,
        grid=grid,
        in_specs=[
            pl.BlockSpec((ET, nbt), lambda b: (b, 0)),
            pl.BlockSpec((ET, 1), lambda b: (b, 0)),
            pl.BlockSpec((_B, 1), lambda b: (0, 0)),
            pl.BlockSpec((1, _EDD), lambda b: (0, 0)),
            pl.BlockSpec((1, _EDD), lambda b: (0, 0)),
            pl.BlockSpec((nbt, _EDD), lambda b: (0, 0)),
            pl.BlockSpec((1, _EDD), lambda b: (0, 0)),
            pl.BlockSpec((_EDD, _EDD), lambda b: (0, 0)),
            pl.BlockSpec((1, _EDD), lambda b: (0, 0)),
        ],
        out_specs=pl.BlockSpec((ET, _EDD), lambda b: (b, 0)),
        out_shape=jax.ShapeDtypeStruct((_E, _EDD), F32),
    )(edge_attr, beg_col, t, p['tmb_W'], p['tmb_b'].reshape(1, -1),
      p['bm_W'], p['bm_b'].reshape(1, -1), p['btm_W'],
      p['btm_b'].reshape(1, -1))


def _tables_body(s_ref, wa_ref, wb_ref, a_ref, b_ref):
    a_ref[...] = _dot(s_ref[...], wa_ref[...])
    b_ref[...] = _dot(s_ref[...], wb_ref[...])


def _node_tables(s, wa, wb):
    return pl.pallas_call(
        _tables_body,
        out_shape=(jax.ShapeDtypeStruct((_N, wa.shape[1]), F32),
                   jax.ShapeDtypeStruct((_N, wb.shape[1]), F32)),
    )(s, wa, wb)


def _edge_pre_body(e_ref, weq_ref, bq_ref, wer_ref, br_ref, q_ref, r_ref):
    e = e_ref[...]
    q_ref[...] = _dot(e, weq_ref[...]) + bq_ref[...]
    r_ref[...] = _dot(e, wer_ref[...]) + br_ref[...]


def _edge_pre(e, weq, bq, wer, br):
    grid = (_E // ET,)
    return pl.pallas_call(
        _edge_pre_body,
        grid=grid,
        in_specs=[
            pl.BlockSpec((ET, _EDD), lambda b: (b, 0)),
            pl.BlockSpec((_EDD, _IND), lambda b: (0, 0)),
            pl.BlockSpec((1, _IND), lambda b: (0, 0)),
            pl.BlockSpec((_EDD, _EDD), lambda b: (0, 0)),
            pl.BlockSpec((1, _EDD), lambda b: (0, 0)),
        ],
        out_specs=(pl.BlockSpec((ET, _IND), lambda b: (b, 0)),
                   pl.BlockSpec((ET, _EDD), lambda b: (b, 0))),
        out_shape=(jax.ShapeDtypeStruct((_E, _IND), F32),
                   jax.ShapeDtypeStruct((_E, _EDD), F32)),
    )(e, weq, bq.reshape(1, -1), wer, br.reshape(1, -1))


def _node_upd_body(s_ref, agg_ref, wns_ref, wna_ref, bn_ref, so_ref):
    s = s_ref[...]
    h = _dot(s, wns_ref[...]) + _dot(agg_ref[...], wna_ref[...]) + bn_ref[...]
    so_ref[...] = s + _silu(h)


def _node_update(s, agg, wns, wna, bn):
    return pl.pallas_call(
        _node_upd_body,
        out_shape=jax.ShapeDtypeStruct((_N, _IND), F32),
    )(s, agg, wns, wna, bn.reshape(1, -1))


def _ewise_body(a_ref, b_ref, c_ref, o_ref):
    o_ref[...] = _silu(a_ref[...] + b_ref[...] + c_ref[...])


def _ewise_silu3(a, b, c):
    w = a.shape[1]
    grid = (_E // ET,)
    spec = pl.BlockSpec((ET, w), lambda g: (g, 0))
    return pl.pallas_call(
        _ewise_body, grid=grid, in_specs=[spec, spec, spec], out_specs=spec,
        out_shape=jax.ShapeDtypeStruct((_E, w), F32),
    )(a, b, c)


def _ewise_res_body(e_ref, a_ref, b_ref, c_ref, o_ref):
    o_ref[...] = e_ref[...] + _silu(a_ref[...] + b_ref[...] + c_ref[...])


def _ewise_residual3(e, a, b, c):
    w = a.shape[1]
    grid = (_E // ET,)
    spec = pl.BlockSpec((ET, w), lambda g: (g, 0))
    return pl.pallas_call(
        _ewise_res_body, grid=grid, in_specs=[spec] * 4, out_specs=spec,
        out_shape=jax.ShapeDtypeStruct((_E, w), F32),
    )(e, a, b, c)


def _head_node_body(s_ref, wsh_ref, bsh_ref, wal_ref, bal_ref,
                    s2_ref, lat_ref, at_ref):
    s2 = _silu(_dot(s_ref[...], wsh_ref[...]) + bsh_ref[...])
    ao = _dot(s2, wal_ref[...]) + bal_ref[...]
    s2_ref[...] = s2
    at_ref[...] = ao[:, :_NAF]
    lat_ref[...] = ao[:, _NAF:]


def _head_node(s, p):
    return pl.pallas_call(
        _head_node_body,
        out_shape=(jax.ShapeDtypeStruct((_N, _IND), F32),
                   jax.ShapeDtypeStruct((_N, _LAT), F32),
                   jax.ShapeDtypeStruct((_N, _NAF), F32)),
    )(s, p['h_sh_W'], p['h_sh_b'].reshape(1, -1),
      p['h_al_W'], p['h_al_b'].reshape(1, -1))


def _head_final_body(g_ref, es_ref, wbm_ref, bbm_ref, wbl_ref, bbl_ref,
                     bo_ref):
    f = _silu(g_ref[...] + _dot(es_ref[...], wbm_ref[...]) + bbm_ref[...])
    bo_ref[...] = _dot(f, wbl_ref[...]) + bbl_ref[...]


def _head_final(gsum, esym, p):
    nbt = p['h_bl_W'].shape[1]
    grid = (_E // ET,)
    return pl.pallas_call(
        _head_final_body,
        grid=grid,
        in_specs=[
            pl.BlockSpec((ET, _IND), lambda b: (b, 0)),
            pl.BlockSpec((ET, _EDD), lambda b: (b, 0)),
            pl.BlockSpec((_EDD, _IND), lambda b: (0, 0)),
            pl.BlockSpec((1, _IND), lambda b: (0, 0)),
            pl.BlockSpec((_IND, nbt), lambda b: (0, 0)),
            pl.BlockSpec((1, nbt), lambda b: (0, 0)),
        ],
        out_specs=pl.BlockSpec((ET, nbt), lambda b: (b, 0)),
        out_shape=jax.ShapeDtypeStruct((_E, nbt), F32),
    )(gsum, esym, p['h_bm_W'], p['h_bm_b'].reshape(1, -1),
      p['h_bl_W'], p['h_bl_b'].reshape(1, -1))


def kernel(x, t, z, edge_attr, params, edge_index, batch, batch_edge_global):
    p = params
    j = edge_index[0]
    i = edge_index[1]
    batch_col = batch.reshape(_N, 1)
    beg_col = batch_edge_global.reshape(_E, 1)

    s = _node_init(x, z, batch_col, t, p)
    e = _edge_init(edge_attr, beg_col, t, p)

    for l in range(_LGNN):
        wmsg = p['gnn_Wmsg'][l]
        wedge = p['gnn_Wedge'][l]
        wnode = p['gnn_Wnode'][l]
        pj, pi = _node_tables(s, wmsg[:_IND], wmsg[_IND:2 * _IND])
        q, r = _edge_pre(e, wmsg[2 * _IND:], p['gnn_bmsg'][l],
                         wedge[2 * _IND:], p['gnn_bedge'][l])
        msg = _ewise_silu3(jnp.take(pj, j, axis=0), jnp.take(pi, i, axis=0), q)
        agg = jax.ops.segment_sum(msg, i, num_segments=_N)
        s = _node_update(s, agg, wnode[:_IND], wnode[_IND:], p['gnn_bnode'][l])
        qj, qi = _node_tables(s, wedge[:_IND], wedge[_IND:2 * _IND])
        e = _ewise_residual3(e, jnp.take(qj, j, axis=0),
                             jnp.take(qi, i, axis=0), r)

    s2, latent_pred, atoms_pred = _head_node(s, p)

    e_dense = jnp.zeros((_N, _N, _EDD), dtype=F32).at[j, i].set(e)
    e_dense = 0.5 * (e_dense + jnp.transpose(e_dense, (1, 0, 2)))
    esym = e_dense[j, i]
    gsum = jnp.take(s2, j, axis=0) + jnp.take(s2, i, axis=0)
    bonds_pred = _head_final(gsum, esym, p)
    return latent_pred, atoms_pred, bonds_pred


# speedup vs baseline: 1.2432x; 1.2432x over previous
import functools

import jax
import jax.numpy as jnp
from jax import lax
from jax.experimental import pallas as pl
from jax.experimental.pallas import tpu as pltpu
from jax.experimental.pallas import tpu_sc as plsc

F32 = jnp.float32
_N = 1024
_E = 131072
_B = 32
_IND = 128
_EDD = 32
_LGNN = 5
_NAF = 16
_LAT = 64
ET = 2048
_HIGH = jax.lax.Precision.DEFAULT


def _dot(a, b):
    return jnp.dot(a, b, precision=_HIGH, preferred_element_type=F32)


def _silu(v):
    return v * jax.nn.sigmoid(v)


def _onehot(idx_col, width):
    t = idx_col.shape[0]
    cols = lax.broadcasted_iota(jnp.int32, (t, width), 1)
    return jnp.where(idx_col == cols, 1.0, 0.0).astype(F32)


def _node_init_body(x_ref, z_ref, b_ref, t_ref, tmaW_ref, tmab_ref,
                    amW_ref, amb_ref, atmW_ref, atmb_ref, lmW_ref, lmb_ref,
                    s_ref):
    ta = _dot(t_ref[...], tmaW_ref[...]) + tmab_ref[...]
    t2 = _dot(ta, atmW_ref[...])
    a1 = _dot(amW_ref[...], atmW_ref[...])
    c2 = _dot(amb_ref[...], atmW_ref[...]) + atmb_ref[...] + lmb_ref[...]
    oh = _onehot(b_ref[...], _B)
    s_ref[...] = (_dot(x_ref[...], a1) + _dot(oh, t2)
                  + _dot(z_ref[...], lmW_ref[...]) + c2)


def _node_init(x, z, batch_col, t, p):
    return pl.pallas_call(
        _node_init_body,
        out_shape=jax.ShapeDtypeStruct((_N, _IND), F32),
    )(x, z, batch_col, t, p['tma_W'], p['tma_b'].reshape(1, -1),
      p['am_W'], p['am_b'].reshape(1, -1), p['atm_W'],
      p['atm_b'].reshape(1, -1), p['lm_W'], p['lm_b'].reshape(1, -1))


def _edge_init_body(ea_ref, beg_ref, t_ref, tmbW_ref, tmbb_ref,
                    bmW_ref, bmb_ref, btmW_ref, btmb_ref, e_ref):
    tb = _dot(t_ref[...], tmbW_ref[...]) + tmbb_ref[...]
    t1 = _dot(tb, btmW_ref[...])
    w1 = _dot(bmW_ref[...], btmW_ref[...])
    c1 = _dot(bmb_ref[...], btmW_ref[...]) + btmb_ref[...]
    oh = _onehot(beg_ref[...], _B)
    e_ref[...] = _dot(ea_ref[...], w1) + _dot(oh, t1) + c1


def _edge_init(edge_attr, beg_col, t, p):
    nbt = edge_attr.shape[1]
    grid = (_E // ET,)
    return pl.pallas_call(
        _edge_init_body,
        grid=grid,
        in_specs=[
            pl.BlockSpec((ET, nbt), lambda b: (b, 0)),
            pl.BlockSpec((ET, 1), lambda b: (b, 0)),
            pl.BlockSpec((_B, 1), lambda b: (0, 0)),
            pl.BlockSpec((1, _EDD), lambda b: (0, 0)),
            pl.BlockSpec((1, _EDD), lambda b: (0, 0)),
            pl.BlockSpec((nbt, _EDD), lambda b: (0, 0)),
            pl.BlockSpec((1, _EDD), lambda b: (0, 0)),
            pl.BlockSpec((_EDD, _EDD), lambda b: (0, 0)),
            pl.BlockSpec((1, _EDD), lambda b: (0, 0)),
        ],
        out_specs=pl.BlockSpec((ET, _EDD), lambda b: (b, 0)),
        out_shape=jax.ShapeDtypeStruct((_E, _EDD), F32),
    )(edge_attr, beg_col, t, p['tmb_W'], p['tmb_b'].reshape(1, -1),
      p['bm_W'], p['bm_b'].reshape(1, -1), p['btm_W'],
      p['btm_b'].reshape(1, -1))


def _tables_body(s_ref, wa_ref, wb_ref, a_ref, b_ref):
    a_ref[...] = _dot(s_ref[...], wa_ref[...])
    b_ref[...] = _dot(s_ref[...], wb_ref[...])


def _node_tables(s, wa, wb):
    return pl.pallas_call(
        _tables_body,
        out_shape=(jax.ShapeDtypeStruct((_N, wa.shape[1]), F32),
                   jax.ShapeDtypeStruct((_N, wb.shape[1]), F32)),
    )(s, wa, wb)


def _edge_pre_body(e_ref, weq_ref, bq_ref, wer_ref, br_ref, q_ref, r_ref):
    e = e_ref[...]
    q_ref[...] = _dot(e, weq_ref[...]) + bq_ref[...]
    r_ref[...] = _dot(e, wer_ref[...]) + br_ref[...]


def _edge_pre(e, weq, bq, wer, br):
    grid = (_E // ET,)
    return pl.pallas_call(
        _edge_pre_body,
        grid=grid,
        in_specs=[
            pl.BlockSpec((ET, _EDD), lambda b: (b, 0)),
            pl.BlockSpec((_EDD, _IND), lambda b: (0, 0)),
            pl.BlockSpec((1, _IND), lambda b: (0, 0)),
            pl.BlockSpec((_EDD, _EDD), lambda b: (0, 0)),
            pl.BlockSpec((1, _EDD), lambda b: (0, 0)),
        ],
        out_specs=(pl.BlockSpec((ET, _IND), lambda b: (b, 0)),
                   pl.BlockSpec((ET, _EDD), lambda b: (b, 0))),
        out_shape=(jax.ShapeDtypeStruct((_E, _IND), F32),
                   jax.ShapeDtypeStruct((_E, _EDD), F32)),
    )(e, weq, bq.reshape(1, -1), wer, br.reshape(1, -1))


_NC = 2
_NS = 16
_NW = _NC * _NS
_EW = _E // _NW
_MC = 128
_MNCH = _EW // _MC


def _sc_msg_body(pj_hbm, pi_hbm, q_hbm, j_hbm, i_hbm, out_hbm,
                 jv, iv, av, bv, qv, zv, agg_sp, sem):
    c = lax.axis_index("c")
    s = lax.axis_index("s")
    w = s * _NC + c
    for cc in range(8):
        for ll in range(8):
            zv[cc, pl.ds(ll * 16, 16)] = jnp.zeros((16,), F32)
    rows_per = _N // _NS
    for k in range(rows_per // 8):
        pltpu.sync_copy(zv, agg_sp.at[pl.ds(s * rows_per + k * 8, 8)])
    pltpu.sync_copy(j_hbm.at[w], jv)
    pltpu.sync_copy(i_hbm.at[w], iv)
    plsc.subcore_barrier()
    for ch in range(_MNCH):
        base = w * _EW + ch * _MC
        pltpu.async_copy(pj_hbm.at[jv.at[ch]], av, sem).wait()
        pltpu.async_copy(pi_hbm.at[iv.at[ch]], bv, sem).wait()
        pltpu.sync_copy(q_hbm.at[pl.ds(base, _MC)], qv)

        def _row(r, carry):
            for cc in range(_IND // 16):
                sl = pl.ds(cc * 16, 16)
                v = av[r, sl] + bv[r, sl] + qv[r, sl]
                av[r, sl] = v / (1.0 + jnp.exp(-v))
            return carry

        lax.fori_loop(0, _MC, _row, 0)
        pltpu.sync_copy(av, agg_sp.at[iv.at[ch]], add=True)
    plsc.subcore_barrier()
    pltpu.sync_copy(agg_sp.at[pl.ds(s * rows_per, rows_per)],
                    out_hbm.at[c, pl.ds(s * rows_per, rows_per)])


@functools.partial(
    pl.kernel,
    mesh=plsc.VectorSubcoreMesh(core_axis_name="c", subcore_axis_name="s"),
    out_type=jax.ShapeDtypeStruct((_NC, _N, _IND), F32),
    scratch_types=[
        pltpu.VMEM((_MNCH, _MC), jnp.int32),
        pltpu.VMEM((_MNCH, _MC), jnp.int32),
        pltpu.VMEM((_MC, _IND), F32),
        pltpu.VMEM((_MC, _IND), F32),
        pltpu.VMEM((_MC, _IND), F32),
        pltpu.VMEM((8, _IND), F32),
        pltpu.VMEM_SHARED((_N, _IND), F32),
        pltpu.SemaphoreType.DMA,
    ],
)
def _sc_msg(pj, pi, q, jr, ir, out, jv, iv, av, bv, qv, zv, agg, sem):
    _sc_msg_body(pj, pi, q, jr, ir, out, jv, iv, av, bv, qv, zv, agg, sem)


def _node_upd_body(s_ref, p0_ref, p1_ref, wns_ref, wna_ref, bn_ref, so_ref):
    s = s_ref[...]
    agg = p0_ref[...] + p1_ref[...]
    h = _dot(s, wns_ref[...]) + _dot(agg, wna_ref[...]) + bn_ref[...]
    so_ref[...] = s + _silu(h)


def _node_update(s, p0, p1, wns, wna, bn):
    return pl.pallas_call(
        _node_upd_body,
        out_shape=jax.ShapeDtypeStruct((_N, _IND), F32),
    )(s, p0, p1, wns, wna, bn.reshape(1, -1))


def _ewise_body(a_ref, b_ref, c_ref, o_ref):
    o_ref[...] = _silu(a_ref[...] + b_ref[...] + c_ref[...])


def _ewise_silu3(a, b, c):
    w = a.shape[1]
    grid = (_E // ET,)
    spec = pl.BlockSpec((ET, w), lambda g: (g, 0))
    return pl.pallas_call(
        _ewise_body, grid=grid, in_specs=[spec, spec, spec], out_specs=spec,
        out_shape=jax.ShapeDtypeStruct((_E, w), F32),
    )(a, b, c)


def _ewise_res_body(e_ref, a_ref, b_ref, c_ref, o_ref):
    o_ref[...] = e_ref[...] + _silu(a_ref[...] + b_ref[...] + c_ref[...])


def _ewise_residual3(e, a, b, c):
    w = a.shape[1]
    grid = (_E // ET,)
    spec = pl.BlockSpec((ET, w), lambda g: (g, 0))
    return pl.pallas_call(
        _ewise_res_body, grid=grid, in_specs=[spec] * 4, out_specs=spec,
        out_shape=jax.ShapeDtypeStruct((_E, w), F32),
    )(e, a, b, c)


def _head_node_body(s_ref, wsh_ref, bsh_ref, wal_ref, bal_ref,
                    s2_ref, lat_ref, at_ref):
    s2 = _silu(_dot(s_ref[...], wsh_ref[...]) + bsh_ref[...])
    ao = _dot(s2, wal_ref[...]) + bal_ref[...]
    s2_ref[...] = s2
    at_ref[...] = ao[:, :_NAF]
    lat_ref[...] = ao[:, _NAF:]


def _head_node(s, p):
    return pl.pallas_call(
        _head_node_body,
        out_shape=(jax.ShapeDtypeStruct((_N, _IND), F32),
                   jax.ShapeDtypeStruct((_N, _LAT), F32),
                   jax.ShapeDtypeStruct((_N, _NAF), F32)),
    )(s, p['h_sh_W'], p['h_sh_b'].reshape(1, -1),
      p['h_al_W'], p['h_al_b'].reshape(1, -1))


def _head_final_body(g_ref, es_ref, wbm_ref, bbm_ref, wbl_ref, bbl_ref,
                     bo_ref):
    f = _silu(g_ref[...] + _dot(es_ref[...], wbm_ref[...]) + bbm_ref[...])
    bo_ref[...] = _dot(f, wbl_ref[...]) + bbl_ref[...]


def _head_final(gsum, esym, p):
    nbt = p['h_bl_W'].shape[1]
    grid = (_E // ET,)
    return pl.pallas_call(
        _head_final_body,
        grid=grid,
        in_specs=[
            pl.BlockSpec((ET, _IND), lambda b: (b, 0)),
            pl.BlockSpec((ET, _EDD), lambda b: (b, 0)),
            pl.BlockSpec((_EDD, _IND), lambda b: (0, 0)),
            pl.BlockSpec((1, _IND), lambda b: (0, 0)),
            pl.BlockSpec((_IND, nbt), lambda b: (0, 0)),
            pl.BlockSpec((1, nbt), lambda b: (0, 0)),
        ],
        out_specs=pl.BlockSpec((ET, nbt), lambda b: (b, 0)),
        out_shape=jax.ShapeDtypeStruct((_E, nbt), F32),
    )(gsum, esym, p['h_bm_W'], p['h_bm_b'].reshape(1, -1),
      p['h_bl_W'], p['h_bl_b'].reshape(1, -1))


def kernel(x, t, z, edge_attr, params, edge_index, batch, batch_edge_global):
    p = params
    j = edge_index[0]
    i = edge_index[1]
    batch_col = batch.reshape(_N, 1)
    beg_col = batch_edge_global.reshape(_E, 1)
    jr3 = j.reshape(_NW, _MNCH, _MC)
    ir3 = i.reshape(_NW, _MNCH, _MC)

    s = _node_init(x, z, batch_col, t, p)
    e = _edge_init(edge_attr, beg_col, t, p)

    for l in range(_LGNN):
        wmsg = p['gnn_Wmsg'][l]
        wedge = p['gnn_Wedge'][l]
        wnode = p['gnn_Wnode'][l]
        pj, pi = _node_tables(s, wmsg[:_IND], wmsg[_IND:2 * _IND])
        q, r = _edge_pre(e, wmsg[2 * _IND:], p['gnn_bmsg'][l],
                         wedge[2 * _IND:], p['gnn_bedge'][l])
        parts = _sc_msg(pj, pi, q, jr3, ir3)
        s = _node_update(s, parts[0], parts[1],
                         wnode[:_IND], wnode[_IND:], p['gnn_bnode'][l])
        qj, qi = _node_tables(s, wedge[:_IND], wedge[_IND:2 * _IND])
        e = _ewise_residual3(e, jnp.take(qj, j, axis=0),
                             jnp.take(qi, i, axis=0), r)

    s2, latent_pred, atoms_pred = _head_node(s, p)

    e_dense = jnp.zeros((_N, _N, _EDD), dtype=F32).at[j, i].set(e)
    e_dense = 0.5 * (e_dense + jnp.transpose(e_dense, (1, 0, 2)))
    esym = e_dense[j, i]
    gsum = jnp.take(s2, j, axis=0) + jnp.take(s2, i, axis=0)
    bonds_pred = _head_final(gsum, esym, p)
    return latent_pred, atoms_pred, bonds_pred


# speedup vs baseline: 1.5172x; 1.2204x over previous
import functools

import jax
import jax.numpy as jnp
from jax import lax
from jax.experimental import pallas as pl
from jax.experimental.pallas import tpu as pltpu
from jax.experimental.pallas import tpu_sc as plsc

F32 = jnp.float32
_N = 1024
_E = 131072
_B = 32
_IND = 128
_EDD = 32
_LGNN = 5
_NAF = 16
_LAT = 64
ET = 2048
_HIGH = jax.lax.Precision.DEFAULT


def _dot(a, b):
    return jnp.dot(a, b, precision=_HIGH, preferred_element_type=F32)


def _silu(v):
    return v * jax.nn.sigmoid(v)


def _onehot(idx_col, width):
    t = idx_col.shape[0]
    cols = lax.broadcasted_iota(jnp.int32, (t, width), 1)
    return jnp.where(idx_col == cols, 1.0, 0.0).astype(F32)


def _node_init_body(x_ref, z_ref, b_ref, t_ref, tmaW_ref, tmab_ref,
                    amW_ref, amb_ref, atmW_ref, atmb_ref, lmW_ref, lmb_ref,
                    s_ref):
    ta = _dot(t_ref[...], tmaW_ref[...]) + tmab_ref[...]
    t2 = _dot(ta, atmW_ref[...])
    a1 = _dot(amW_ref[...], atmW_ref[...])
    c2 = _dot(amb_ref[...], atmW_ref[...]) + atmb_ref[...] + lmb_ref[...]
    oh = _onehot(b_ref[...], _B)
    s_ref[...] = (_dot(x_ref[...], a1) + _dot(oh, t2)
                  + _dot(z_ref[...], lmW_ref[...]) + c2)


def _node_init(x, z, batch_col, t, p):
    return pl.pallas_call(
        _node_init_body,
        out_shape=jax.ShapeDtypeStruct((_N, _IND), F32),
    )(x, z, batch_col, t, p['tma_W'], p['tma_b'].reshape(1, -1),
      p['am_W'], p['am_b'].reshape(1, -1), p['atm_W'],
      p['atm_b'].reshape(1, -1), p['lm_W'], p['lm_b'].reshape(1, -1))


def _edge_init_body(ea_ref, beg_ref, t_ref, tmbW_ref, tmbb_ref,
                    bmW_ref, bmb_ref, btmW_ref, btmb_ref, e_ref):
    tb = _dot(t_ref[...], tmbW_ref[...]) + tmbb_ref[...]
    t1 = _dot(tb, btmW_ref[...])
    w1 = _dot(bmW_ref[...], btmW_ref[...])
    c1 = _dot(bmb_ref[...], btmW_ref[...]) + btmb_ref[...]
    oh = _onehot(beg_ref[...], _B)
    e_ref[...] = _dot(ea_ref[...], w1) + _dot(oh, t1) + c1


def _edge_init(edge_attr, beg_col, t, p):
    nbt = edge_attr.shape[1]
    grid = (_E // ET,)
    return pl.pallas_call(
        _edge_init_body,
        grid=grid,
        in_specs=[
            pl.BlockSpec((ET, nbt), lambda b: (b, 0)),
            pl.BlockSpec((ET, 1), lambda b: (b, 0)),
            pl.BlockSpec((_B, 1), lambda b: (0, 0)),
            pl.BlockSpec((1, _EDD), lambda b: (0, 0)),
            pl.BlockSpec((1, _EDD), lambda b: (0, 0)),
            pl.BlockSpec((nbt, _EDD), lambda b: (0, 0)),
            pl.BlockSpec((1, _EDD), lambda b: (0, 0)),
            pl.BlockSpec((_EDD, _EDD), lambda b: (0, 0)),
            pl.BlockSpec((1, _EDD), lambda b: (0, 0)),
        ],
        out_specs=pl.BlockSpec((ET, _EDD), lambda b: (b, 0)),
        out_shape=jax.ShapeDtypeStruct((_E, _EDD), F32),
    )(edge_attr, beg_col, t, p['tmb_W'], p['tmb_b'].reshape(1, -1),
      p['bm_W'], p['bm_b'].reshape(1, -1), p['btm_W'],
      p['btm_b'].reshape(1, -1))


def _tables_body(s_ref, wa_ref, wb_ref, a_ref, b_ref):
    a_ref[...] = _dot(s_ref[...], wa_ref[...])
    b_ref[...] = _dot(s_ref[...], wb_ref[...])


def _node_tables(s, wa, wb):
    return pl.pallas_call(
        _tables_body,
        out_shape=(jax.ShapeDtypeStruct((_N, wa.shape[1]), F32),
                   jax.ShapeDtypeStruct((_N, wb.shape[1]), F32)),
    )(s, wa, wb)


def _combo_body(s_ref, wa_ref, wb_ref, o_ref):
    s = s_ref[...]
    z = jnp.zeros((_N, _IND - 2 * _EDD), F32)
    o_ref[...] = jnp.concatenate(
        [_dot(s, wa_ref[...]), _dot(s, wb_ref[...]), z], axis=1)


def _combo_table(s, wa, wb):
    return pl.pallas_call(
        _combo_body,
        out_shape=jax.ShapeDtypeStruct((_N, _IND), F32),
    )(s, wa, wb)


def _edge_pre_body(e_ref, weq_ref, bq_ref, wer_ref, br_ref, q_ref, r_ref):
    e = e_ref[...]
    q_ref[...] = _dot(e, weq_ref[...]) + bq_ref[...]
    r_ref[...] = _dot(e, wer_ref[...]) + br_ref[...]


def _edge_pre(e, weq, bq, wer, br):
    grid = (_E // ET,)
    return pl.pallas_call(
        _edge_pre_body,
        grid=grid,
        in_specs=[
            pl.BlockSpec((ET, _EDD), lambda b: (b, 0)),
            pl.BlockSpec((_EDD, _IND), lambda b: (0, 0)),
            pl.BlockSpec((1, _IND), lambda b: (0, 0)),
            pl.BlockSpec((_EDD, _EDD), lambda b: (0, 0)),
            pl.BlockSpec((1, _EDD), lambda b: (0, 0)),
        ],
        out_specs=(pl.BlockSpec((ET, _IND), lambda b: (b, 0)),
                   pl.BlockSpec((ET, _EDD), lambda b: (b, 0))),
        out_shape=(jax.ShapeDtypeStruct((_E, _IND), F32),
                   jax.ShapeDtypeStruct((_E, _EDD), F32)),
    )(e, weq, bq.reshape(1, -1), wer, br.reshape(1, -1))


_NC = 2
_NS = 16
_NW = _NC * _NS
_EW = _E // _NW
_MC = 128
_MNCH = _EW // _MC


def _sc_msg_body(pj_hbm, pi_hbm, q_hbm, j_hbm, i_hbm, out_hbm,
                 jv, iv, av, bv, qv, zv, agg_sp, sem):
    c = lax.axis_index("c")
    s = lax.axis_index("s")
    w = s * _NC + c
    for cc in range(8):
        for ll in range(8):
            zv[cc, pl.ds(ll * 16, 16)] = jnp.zeros((16,), F32)
    rows_per = _N // _NS
    for k in range(rows_per // 8):
        pltpu.sync_copy(zv, agg_sp.at[pl.ds(s * rows_per + k * 8, 8)])
    pltpu.sync_copy(j_hbm.at[w], jv)
    pltpu.sync_copy(i_hbm.at[w], iv)
    plsc.subcore_barrier()
    for ch in range(_MNCH):
        base = w * _EW + ch * _MC
        pltpu.async_copy(pj_hbm.at[jv.at[ch]], av, sem).wait()
        pltpu.async_copy(pi_hbm.at[iv.at[ch]], bv, sem).wait()
        pltpu.sync_copy(q_hbm.at[pl.ds(base, _MC)], qv)

        def _row(r, carry):
            for cc in range(_IND // 16):
                sl = pl.ds(cc * 16, 16)
                v = av[r, sl] + bv[r, sl] + qv[r, sl]
                av[r, sl] = v / (1.0 + jnp.exp(-v))
            return carry

        lax.fori_loop(0, _MC, _row, 0)
        pltpu.sync_copy(av, agg_sp.at[iv.at[ch]], add=True)
    plsc.subcore_barrier()
    pltpu.sync_copy(agg_sp.at[pl.ds(s * rows_per, rows_per)],
                    out_hbm.at[c, pl.ds(s * rows_per, rows_per)])


@functools.partial(
    pl.kernel,
    mesh=plsc.VectorSubcoreMesh(core_axis_name="c", subcore_axis_name="s"),
    out_type=jax.ShapeDtypeStruct((_NC, _N, _IND), F32),
    scratch_types=[
        pltpu.VMEM((_MNCH, _MC), jnp.int32),
        pltpu.VMEM((_MNCH, _MC), jnp.int32),
        pltpu.VMEM((_MC, _IND), F32),
        pltpu.VMEM((_MC, _IND), F32),
        pltpu.VMEM((_MC, _IND), F32),
        pltpu.VMEM((8, _IND), F32),
        pltpu.VMEM_SHARED((_N, _IND), F32),
        pltpu.SemaphoreType.DMA,
    ],
)
def _sc_msg(pj, pi, q, jr, ir, out, jv, iv, av, bv, qv, zv, agg, sem):
    _sc_msg_body(pj, pi, q, jr, ir, out, jv, iv, av, bv, qv, zv, agg, sem)


def _sc_edge_body(qt_hbm, eo_hbm, r_hbm, j_hbm, i_hbm, out_hbm,
                  jv, iv, av, bv, ev, rv, sem):
    c = lax.axis_index("c")
    s = lax.axis_index("s")
    w = s * _NC + c
    pltpu.sync_copy(j_hbm.at[w], jv)
    pltpu.sync_copy(i_hbm.at[w], iv)
    for ch in range(_MNCH):
        base = w * _EW + ch * _MC
        pltpu.async_copy(qt_hbm.at[jv.at[ch]], av, sem).wait()
        pltpu.async_copy(qt_hbm.at[iv.at[ch]], bv, sem).wait()
        pltpu.sync_copy(eo_hbm.at[pl.ds(base, _MC)], ev)
        pltpu.sync_copy(r_hbm.at[pl.ds(base, _MC)], rv)

        def _row(rr, carry):
            for cc in range(_EDD // 16):
                sl = pl.ds(cc * 16, 16)
                v = (av[rr, sl] + bv[rr, pl.ds(_EDD + cc * 16, 16)]
                     + rv[rr, sl])
                ev[rr, sl] = ev[rr, sl] + v / (1.0 + jnp.exp(-v))
            return carry

        lax.fori_loop(0, _MC, _row, 0)
        pltpu.sync_copy(ev, out_hbm.at[pl.ds(base, _MC)])


@functools.partial(
    pl.kernel,
    mesh=plsc.VectorSubcoreMesh(core_axis_name="c", subcore_axis_name="s"),
    out_type=jax.ShapeDtypeStruct((_E, _EDD), F32),
    scratch_types=[
        pltpu.VMEM((_MNCH, _MC), jnp.int32),
        pltpu.VMEM((_MNCH, _MC), jnp.int32),
        pltpu.VMEM((_MC, _IND), F32),
        pltpu.VMEM((_MC, _IND), F32),
        pltpu.VMEM((_MC, _EDD), F32),
        pltpu.VMEM((_MC, _EDD), F32),
        pltpu.SemaphoreType.DMA,
    ],
)
def _sc_edge(qt, e, r, jr, ir, out, jv, iv, av, bv, ev, rv, sem):
    _sc_edge_body(qt, e, r, jr, ir, out, jv, iv, av, bv, ev, rv, sem)


def _node_upd_body(s_ref, p0_ref, p1_ref, wns_ref, wna_ref, bn_ref, so_ref):
    s = s_ref[...]
    agg = p0_ref[...] + p1_ref[...]
    h = _dot(s, wns_ref[...]) + _dot(agg, wna_ref[...]) + bn_ref[...]
    so_ref[...] = s + _silu(h)


def _node_update(s, p0, p1, wns, wna, bn):
    return pl.pallas_call(
        _node_upd_body,
        out_shape=jax.ShapeDtypeStruct((_N, _IND), F32),
    )(s, p0, p1, wns, wna, bn.reshape(1, -1))


def _ewise_body(a_ref, b_ref, c_ref, o_ref):
    o_ref[...] = _silu(a_ref[...] + b_ref[...] + c_ref[...])


def _ewise_silu3(a, b, c):
    w = a.shape[1]
    grid = (_E // ET,)
    spec = pl.BlockSpec((ET, w), lambda g: (g, 0))
    return pl.pallas_call(
        _ewise_body, grid=grid, in_specs=[spec, spec, spec], out_specs=spec,
        out_shape=jax.ShapeDtypeStruct((_E, w), F32),
    )(a, b, c)


def _ewise_res_body(e_ref, a_ref, b_ref, c_ref, o_ref):
    o_ref[...] = e_ref[...] + _silu(a_ref[...] + b_ref[...] + c_ref[...])


def _ewise_residual3(e, a, b, c):
    w = a.shape[1]
    grid = (_E // ET,)
    spec = pl.BlockSpec((ET, w), lambda g: (g, 0))
    return pl.pallas_call(
        _ewise_res_body, grid=grid, in_specs=[spec] * 4, out_specs=spec,
        out_shape=jax.ShapeDtypeStruct((_E, w), F32),
    )(e, a, b, c)


def _head_node_body(s_ref, wsh_ref, bsh_ref, wal_ref, bal_ref,
                    s2_ref, lat_ref, at_ref):
    s2 = _silu(_dot(s_ref[...], wsh_ref[...]) + bsh_ref[...])
    ao = _dot(s2, wal_ref[...]) + bal_ref[...]
    s2_ref[...] = s2
    at_ref[...] = ao[:, :_NAF]
    lat_ref[...] = ao[:, _NAF:]


def _head_node(s, p):
    return pl.pallas_call(
        _head_node_body,
        out_shape=(jax.ShapeDtypeStruct((_N, _IND), F32),
                   jax.ShapeDtypeStruct((_N, _LAT), F32),
                   jax.ShapeDtypeStruct((_N, _NAF), F32)),
    )(s, p['h_sh_W'], p['h_sh_b'].reshape(1, -1),
      p['h_al_W'], p['h_al_b'].reshape(1, -1))


def _head_final_body(g_ref, es_ref, wbm_ref, bbm_ref, wbl_ref, bbl_ref,
                     bo_ref):
    f = _silu(g_ref[...] + _dot(es_ref[...], wbm_ref[...]) + bbm_ref[...])
    bo_ref[...] = _dot(f, wbl_ref[...]) + bbl_ref[...]


def _head_final(gsum, esym, p):
    nbt = p['h_bl_W'].shape[1]
    grid = (_E // ET,)
    return pl.pallas_call(
        _head_final_body,
        grid=grid,
        in_specs=[
            pl.BlockSpec((ET, _IND), lambda b: (b, 0)),
            pl.BlockSpec((ET, _EDD), lambda b: (b, 0)),
            pl.BlockSpec((_EDD, _IND), lambda b: (0, 0)),
            pl.BlockSpec((1, _IND), lambda b: (0, 0)),
            pl.BlockSpec((_IND, nbt), lambda b: (0, 0)),
            pl.BlockSpec((1, nbt), lambda b: (0, 0)),
        ],
        out_specs=pl.BlockSpec((ET, nbt), lambda b: (b, 0)),
        out_shape=jax.ShapeDtypeStruct((_E, nbt), F32),
    )(gsum, esym, p['h_bm_W'], p['h_bm_b'].reshape(1, -1),
      p['h_bl_W'], p['h_bl_b'].reshape(1, -1))


def kernel(x, t, z, edge_attr, params, edge_index, batch, batch_edge_global):
    p = params
    j = edge_index[0]
    i = edge_index[1]
    batch_col = batch.reshape(_N, 1)
    beg_col = batch_edge_global.reshape(_E, 1)
    jr3 = j.reshape(_NW, _MNCH, _MC)
    ir3 = i.reshape(_NW, _MNCH, _MC)

    s = _node_init(x, z, batch_col, t, p)
    e = _edge_init(edge_attr, beg_col, t, p)

    for l in range(_LGNN):
        wmsg = p['gnn_Wmsg'][l]
        wedge = p['gnn_Wedge'][l]
        wnode = p['gnn_Wnode'][l]
        pj, pi = _node_tables(s, wmsg[:_IND], wmsg[_IND:2 * _IND])
        q, r = _edge_pre(e, wmsg[2 * _IND:], p['gnn_bmsg'][l],
                         wedge[2 * _IND:], p['gnn_bedge'][l])
        parts = _sc_msg(pj, pi, q, jr3, ir3)
        s = _node_update(s, parts[0], parts[1],
                         wnode[:_IND], wnode[_IND:], p['gnn_bnode'][l])
        qt = _combo_table(s, wedge[:_IND], wedge[_IND:2 * _IND])
        e = _sc_edge(qt, e, r, jr3, ir3)

    s2, latent_pred, atoms_pred = _head_node(s, p)

    e_dense = jnp.zeros((_N, _N, _EDD), dtype=F32).at[j, i].set(e)
    e_dense = 0.5 * (e_dense + jnp.transpose(e_dense, (1, 0, 2)))
    esym = e_dense[j, i]
    gsum = jnp.take(s2, j, axis=0) + jnp.take(s2, i, axis=0)
    bonds_pred = _head_final(gsum, esym, p)
    return latent_pred, atoms_pred, bonds_pred


# speedup vs baseline: 3.3688x; 2.2204x over previous
import functools

import jax
import jax.numpy as jnp
from jax import lax
from jax.experimental import pallas as pl
from jax.experimental.pallas import tpu as pltpu
from jax.experimental.pallas import tpu_sc as plsc

F32 = jnp.float32
I32 = jnp.int32
_N = 1024
_E = 131072
_B = 32
_IND = 128
_EDD = 32
_LGNN = 5
_NAF = 16
_LAT = 64
ET = 2048
_HIGH = jax.lax.Precision.DEFAULT


def _dot(a, b):
    return jnp.dot(a, b, precision=_HIGH, preferred_element_type=F32)


def _silu(v):
    return v * jax.nn.sigmoid(v)


def _onehot(idx_col, width):
    t = idx_col.shape[0]
    cols = lax.broadcasted_iota(I32, (t, width), 1)
    return jnp.where(idx_col == cols, 1.0, 0.0).astype(F32)


def _node_init_body(x_ref, z_ref, b_ref, t_ref, tmaW_ref, tmab_ref,
                    amW_ref, amb_ref, atmW_ref, atmb_ref, lmW_ref, lmb_ref,
                    s_ref):
    ta = _dot(t_ref[...], tmaW_ref[...]) + tmab_ref[...]
    t2 = _dot(ta, atmW_ref[...])
    a1 = _dot(amW_ref[...], atmW_ref[...])
    c2 = _dot(amb_ref[...], atmW_ref[...]) + atmb_ref[...] + lmb_ref[...]
    oh = _onehot(b_ref[...], _B)
    s_ref[...] = (_dot(x_ref[...], a1) + _dot(oh, t2)
                  + _dot(z_ref[...], lmW_ref[...]) + c2)


def _node_init(x, z, batch_col, t, p):
    return pl.pallas_call(
        _node_init_body,
        out_shape=jax.ShapeDtypeStruct((_N, _IND), F32),
    )(x, z, batch_col, t, p['tma_W'], p['tma_b'].reshape(1, -1),
      p['am_W'], p['am_b'].reshape(1, -1), p['atm_W'],
      p['atm_b'].reshape(1, -1), p['lm_W'], p['lm_b'].reshape(1, -1))


def _edge_init_body(ea_ref, beg_ref, t_ref, tmbW_ref, tmbb_ref,
                    bmW_ref, bmb_ref, btmW_ref, btmb_ref, e_ref):
    tb = _dot(t_ref[...], tmbW_ref[...]) + tmbb_ref[...]
    t1 = _dot(tb, btmW_ref[...])
    w1 = _dot(bmW_ref[...], btmW_ref[...])
    c1 = _dot(bmb_ref[...], btmW_ref[...]) + btmb_ref[...]
    oh = _onehot(beg_ref[...], _B)
    e_ref[...] = _dot(ea_ref[...], w1) + _dot(oh, t1) + c1


def _edge_init(edge_attr, beg_col, t, p):
    nbt = edge_attr.shape[1]
    grid = (_E // ET,)
    return pl.pallas_call(
        _edge_init_body,
        grid=grid,
        in_specs=[
            pl.BlockSpec((ET, nbt), lambda b: (b, 0)),
            pl.BlockSpec((ET, 1), lambda b: (b, 0)),
            pl.BlockSpec((_B, 1), lambda b: (0, 0)),
            pl.BlockSpec((1, _EDD), lambda b: (0, 0)),
            pl.BlockSpec((1, _EDD), lambda b: (0, 0)),
            pl.BlockSpec((nbt, _EDD), lambda b: (0, 0)),
            pl.BlockSpec((1, _EDD), lambda b: (0, 0)),
            pl.BlockSpec((_EDD, _EDD), lambda b: (0, 0)),
            pl.BlockSpec((1, _EDD), lambda b: (0, 0)),
        ],
        out_specs=pl.BlockSpec((ET, _EDD), lambda b: (b, 0)),
        out_shape=jax.ShapeDtypeStruct((_E, _EDD), F32),
    )(edge_attr, beg_col, t, p['tmb_W'], p['tmb_b'].reshape(1, -1),
      p['bm_W'], p['bm_b'].reshape(1, -1), p['btm_W'],
      p['btm_b'].reshape(1, -1))


def _tables_body(s_ref, wa_ref, wb_ref, a_ref, b_ref):
    a_ref[...] = _dot(s_ref[...], wa_ref[...])
    b_ref[...] = _dot(s_ref[...], wb_ref[...])


def _node_tables(s, wa, wb):
    return pl.pallas_call(
        _tables_body,
        out_shape=(jax.ShapeDtypeStruct((_N, wa.shape[1]), F32),
                   jax.ShapeDtypeStruct((_N, wb.shape[1]), F32)),
    )(s, wa, wb)


def _combo_body(s_ref, wa_ref, wb_ref, o_ref):
    s = s_ref[...]
    z = jnp.zeros((_N, _IND - 2 * _EDD), F32)
    o_ref[...] = jnp.concatenate(
        [_dot(s, wa_ref[...]), _dot(s, wb_ref[...]), z], axis=1)


def _combo_table(s, wa, wb):
    return pl.pallas_call(
        _combo_body,
        out_shape=jax.ShapeDtypeStruct((_N, _IND), F32),
    )(s, wa, wb)


def _edge_pre_body(e_ref, weq_ref, bq_ref, wer_ref, br_ref, q_ref, r_ref):
    e = e_ref[...]
    q_ref[...] = _dot(e, weq_ref[...]) + bq_ref[...]
    r_ref[...] = _dot(e, wer_ref[...]) + br_ref[...]


def _edge_pre(e, weq, bq, wer, br):
    grid = (_E // ET,)
    return pl.pallas_call(
        _edge_pre_body,
        grid=grid,
        in_specs=[
            pl.BlockSpec((ET, _EDD), lambda b: (b, 0)),
            pl.BlockSpec((_EDD, _IND), lambda b: (0, 0)),
            pl.BlockSpec((1, _IND), lambda b: (0, 0)),
            pl.BlockSpec((_EDD, _EDD), lambda b: (0, 0)),
            pl.BlockSpec((1, _EDD), lambda b: (0, 0)),
        ],
        out_specs=(pl.BlockSpec((ET, _IND), lambda b: (b, 0)),
                   pl.BlockSpec((ET, _EDD), lambda b: (b, 0))),
        out_shape=(jax.ShapeDtypeStruct((_E, _IND), F32),
                   jax.ShapeDtypeStruct((_E, _EDD), F32)),
    )(e, weq, bq.reshape(1, -1), wer, br.reshape(1, -1))


_NC = 2
_NS = 16
_NW = _NC * _NS
_EW = _E // _NW
_MC = 128
_MNCH = _EW // _MC
_EPAD = 2048


def _sc_msg_body(pj_hbm, pi_hbm, q_hbm, j_hbm, i_hbm, out_hbm,
                 jv, iv, av, bv, qv, zv, agg_sp, sem):
    c = lax.axis_index("c")
    s = lax.axis_index("s")
    w = s * _NC + c
    for cc in range(8):
        for ll in range(8):
            zv[cc, pl.ds(ll * 16, 16)] = jnp.zeros((16,), F32)
    rows_per = _N // _NS
    for k in range(rows_per // 8):
        pltpu.sync_copy(zv, agg_sp.at[pl.ds(s * rows_per + k * 8, 8)])
    pltpu.sync_copy(j_hbm.at[w], jv)
    pltpu.sync_copy(i_hbm.at[w], iv)
    plsc.subcore_barrier()
    for ch in range(_MNCH):
        base = w * _EW + ch * _MC
        pltpu.async_copy(pj_hbm.at[jv.at[ch]], av, sem).wait()
        pltpu.async_copy(pi_hbm.at[iv.at[ch]], bv, sem).wait()
        pltpu.sync_copy(q_hbm.at[pl.ds(base, _MC)], qv)

        def _row(r, carry):
            for cc in range(_IND // 16):
                sl = pl.ds(cc * 16, 16)
                v = av[r, sl] + bv[r, sl] + qv[r, sl]
                av[r, sl] = v / (1.0 + jnp.exp(-v))
            return carry

        lax.fori_loop(0, _MC, _row, 0)
        pltpu.sync_copy(av, agg_sp.at[iv.at[ch]], add=True)
    plsc.subcore_barrier()
    pltpu.sync_copy(agg_sp.at[pl.ds(s * rows_per, rows_per)],
                    out_hbm.at[c, pl.ds(s * rows_per, rows_per)])


@functools.partial(
    pl.kernel,
    mesh=plsc.VectorSubcoreMesh(core_axis_name="c", subcore_axis_name="s"),
    out_type=jax.ShapeDtypeStruct((_NC, _N, _IND), F32),
    scratch_types=[
        pltpu.VMEM((_MNCH, _MC), I32),
        pltpu.VMEM((_MNCH, _MC), I32),
        pltpu.VMEM((_MC, _IND), F32),
        pltpu.VMEM((_MC, _IND), F32),
        pltpu.VMEM((_MC, _IND), F32),
        pltpu.VMEM((8, _IND), F32),
        pltpu.VMEM_SHARED((_N, _IND), F32),
        pltpu.SemaphoreType.DMA,
    ],
)
def _sc_msg(pj, pi, q, jr, ir, out, jv, iv, av, bv, qv, zv, agg, sem):
    _sc_msg_body(pj, pi, q, jr, ir, out, jv, iv, av, bv, qv, zv, agg, sem)


def _sc_edge_common(qt_hbm, eo_hbm, r_hbm, j_hbm, i_hbm, out_hbm,
                    jv, iv, av, bv, ev, rv, sem, et=None):
    c = lax.axis_index("c")
    s = lax.axis_index("s")
    w = s * _NC + c
    pad = et is not None
    pltpu.sync_copy(j_hbm.at[w], jv)
    pltpu.sync_copy(i_hbm.at[w], iv)
    if pad:
        def _zr(r, carry):
            for cc in range(_IND // 16):
                ev[r, pl.ds(cc * 16, 16)] = jnp.zeros((16,), F32)
            return carry

        lax.fori_loop(0, _MC, _zr, 0)

        @pl.when(w == 0)
        def _pad_rows():
            def _pz(k, carry):
                pltpu.sync_copy(ev, out_hbm.at[pl.ds(_E + k * _MC, _MC)])
                return carry

            lax.fori_loop(0, _EPAD // _MC, _pz, 0)
    for ch in range(_MNCH):
        base = w * _EW + ch * _MC
        pltpu.async_copy(qt_hbm.at[jv.at[ch]], av, sem).wait()
        pltpu.async_copy(qt_hbm.at[iv.at[ch]], bv, sem).wait()
        pltpu.sync_copy(eo_hbm.at[pl.ds(base, _MC)], et if pad else ev)
        pltpu.sync_copy(r_hbm.at[pl.ds(base, _MC)], rv)

        def _row(rr, carry):
            for cc in range(_EDD // 16):
                sl = pl.ds(cc * 16, 16)
                v = (av[rr, sl] + bv[rr, pl.ds(_EDD + cc * 16, 16)]
                     + rv[rr, sl])
                eold = et[rr, sl] if pad else ev[rr, sl]
                ev[rr, sl] = eold + v / (1.0 + jnp.exp(-v))
            return carry

        lax.fori_loop(0, _MC, _row, 0)
        pltpu.sync_copy(ev, out_hbm.at[pl.ds(base, _MC)])


@functools.partial(
    pl.kernel,
    mesh=plsc.VectorSubcoreMesh(core_axis_name="c", subcore_axis_name="s"),
    out_type=jax.ShapeDtypeStruct((_E, _EDD), F32),
    scratch_types=[
        pltpu.VMEM((_MNCH, _MC), I32),
        pltpu.VMEM((_MNCH, _MC), I32),
        pltpu.VMEM((_MC, _IND), F32),
        pltpu.VMEM((_MC, _IND), F32),
        pltpu.VMEM((_MC, _EDD), F32),
        pltpu.VMEM((_MC, _EDD), F32),
        pltpu.SemaphoreType.DMA,
    ],
)
def _sc_edge(qt, e, r, jr, ir, out, jv, iv, av, bv, ev, rv, sem):
    _sc_edge_common(qt, e, r, jr, ir, out, jv, iv, av, bv, ev, rv, sem)


@functools.partial(
    pl.kernel,
    mesh=plsc.VectorSubcoreMesh(core_axis_name="c", subcore_axis_name="s"),
    out_type=jax.ShapeDtypeStruct((_E + _EPAD, _IND), F32),
    scratch_types=[
        pltpu.VMEM((_MNCH, _MC), I32),
        pltpu.VMEM((_MNCH, _MC), I32),
        pltpu.VMEM((_MC, _IND), F32),
        pltpu.VMEM((_MC, _IND), F32),
        pltpu.VMEM((_MC, _IND), F32),
        pltpu.VMEM((_MC, _EDD), F32),
        pltpu.VMEM((_MC, _EDD), F32),
        pltpu.SemaphoreType.DMA,
    ],
)
def _sc_edge_pad(qt, e, r, jr, ir, out, jv, iv, av, bv, ev, rv, et, sem):
    _sc_edge_common(qt, e, r, jr, ir, out, jv, iv, av, bv, ev, rv, sem,
                    et=et)


_PN = _N * _N
_PTOT = _PN + 16384
_ZW = _PTOT // _NS
_FIXR = 5


def _sc_head1_body(s2_hbm, j_hbm, i_hbm, p_hbm, g_hbm,
                   jv64, iv64, kx, vb, pv4, k2, zb, av, bv, sem, sem2):
    c = lax.axis_index("c")
    s = lax.axis_index("s")

    @pl.when(c == 0)
    def _build_p():
        pltpu.sync_copy(j_hbm.at[2 * s], jv64.at[pl.ds(0, 32)])
        pltpu.sync_copy(j_hbm.at[2 * s + 1], jv64.at[pl.ds(32, 32)])
        pltpu.sync_copy(i_hbm.at[2 * s], iv64.at[pl.ds(0, 32)])
        pltpu.sync_copy(i_hbm.at[2 * s + 1], iv64.at[pl.ds(32, 32)])
        for g in range(64):
            zb[pl.ds(g * 16, 16)] = jnp.zeros((16,), I32)

        def _z(k, carry):
            pltpu.sync_copy(zb, p_hbm.at[pl.ds(s * _ZW + k * 1024, 1024)])
            return carry

        lax.fori_loop(0, _ZW // 1024, _z, 0)

        def _kv(ch, carry):
            for g in range(8):
                sl = pl.ds(g * 16, 16)
                kx[ch, sl] = jv64[ch, sl] * _N + iv64[ch, sl]
                vb[ch, sl] = (s * 8192 + ch * 128 + g * 16 + 1
                              + lax.iota(I32, 16))
            return carry

        lax.fori_loop(0, 64, _kv, 0)
        plsc.subcore_barrier()

        def _r1(c2, carry):
            hs = [pltpu.async_copy(vb.at[c2 * 4 + u], p_hbm.at[kx.at[c2 * 4 + u]],
                                   sem) for u in range(4)]
            for h in hs:
                h.wait()
            return carry

        lax.fori_loop(0, 16, _r1, 0)
        plsc.subcore_barrier()
        for _ in range(_FIXR):
            def _fr(c2, carry):
                hs = [pltpu.async_copy(p_hbm.at[kx.at[c2 * 4 + u]], pv4.at[u],
                                       sem) for u in range(4)]
                for h in hs:
                    h.wait()
                for u in range(4):
                    ch = c2 * 4 + u
                    for g in range(8):
                        sl = pl.ds(g * 16, 16)
                        mywin = vb[ch, sl] > pv4[u, sl]
                        dump = (_PN + ch * 128 + g * 16
                                + lax.iota(I32, 16))
                        k2[u, sl] = jnp.where(mywin, kx[ch, sl], dump)
                hs2 = [pltpu.async_copy(vb.at[c2 * 4 + u], p_hbm.at[k2.at[u]],
                                        sem) for u in range(4)]
                for h in hs2:
                    h.wait()
                return carry

            lax.fori_loop(0, 16, _fr, 0)
            plsc.subcore_barrier()

    @pl.when(c == 1)
    def _gsum():
        pltpu.sync_copy(j_hbm.at[2 * s], jv64.at[pl.ds(0, 32)])
        pltpu.sync_copy(j_hbm.at[2 * s + 1], jv64.at[pl.ds(32, 32)])
        pltpu.sync_copy(i_hbm.at[2 * s], iv64.at[pl.ds(0, 32)])
        pltpu.sync_copy(i_hbm.at[2 * s + 1], iv64.at[pl.ds(32, 32)])

        def _gs(ch, carry):
            pltpu.async_copy(s2_hbm.at[jv64.at[ch]], av, sem2).wait()
            pltpu.async_copy(s2_hbm.at[iv64.at[ch]], bv, sem2).wait()

            def _row(r, carry2):
                for g in range(8):
                    sl = pl.ds(g * 16, 16)
                    av[r, sl] = av[r, sl] + bv[r, sl]
                return carry2

            lax.fori_loop(0, 128, _row, 0)
            pltpu.sync_copy(av, g_hbm.at[pl.ds(s * 8192 + ch * 128, 128)])
            return carry

        lax.fori_loop(0, 64, _gs, 0)


@functools.partial(
    pl.kernel,
    mesh=plsc.VectorSubcoreMesh(core_axis_name="c", subcore_axis_name="s"),
    out_type=(jax.ShapeDtypeStruct((_PTOT,), I32),
              jax.ShapeDtypeStruct((_E, _IND), F32)),
    scratch_types=[
        pltpu.VMEM((64, 128), I32),
        pltpu.VMEM((64, 128), I32),
        pltpu.VMEM((64, 128), I32),
        pltpu.VMEM((64, 128), I32),
        pltpu.VMEM((4, 128), I32),
        pltpu.VMEM((4, 128), I32),
        pltpu.VMEM((1024,), I32),
        pltpu.VMEM((128, 128), F32),
        pltpu.VMEM((128, 128), F32),
        pltpu.SemaphoreType.DMA,
        pltpu.SemaphoreType.DMA,
    ],
)
def _sc_head1(s2, jr, ir, p_out, g_out,
              jv64, iv64, kx, vb, pv4, k2, zb, av, bv, sem, sem2):
    _sc_head1_body(s2, jr, ir, p_out, g_out,
                   jv64, iv64, kx, vb, pv4, k2, zb, av, bv, sem, sem2)


def _sc_head2_body(p_hbm, ep_hbm, j_hbm, i_hbm, es_hbm,
                   jv, iv, kb, rb, z1, z2, i1, i2, av, bv, ev, sem):
    c = lax.axis_index("c")
    s = lax.axis_index("s")
    w = s * _NC + c
    pltpu.sync_copy(j_hbm.at[w], jv)
    pltpu.sync_copy(i_hbm.at[w], iv)

    def _ch(ch, carry):
        base = w * _EW + ch * _MC
        for g in range(8):
            sl = pl.ds(g * 16, 16)
            kb[sl] = jv[ch, sl] * _N + iv[ch, sl]
            rb[sl] = iv[ch, sl] * _N + jv[ch, sl]
        pltpu.async_copy(p_hbm.at[kb], z1, sem).wait()
        pltpu.async_copy(p_hbm.at[rb], z2, sem).wait()
        for g in range(8):
            sl = pl.ds(g * 16, 16)
            i1[sl] = z1[sl] - 1
            dummy = _E + ((kb[sl] + g * 16 + lax.iota(I32, 16)) & (_EPAD - 1))
            i2[sl] = jnp.where(z2[sl] > 0, z2[sl] - 1, dummy)
        pltpu.async_copy(ep_hbm.at[i1], av, sem).wait()
        pltpu.async_copy(ep_hbm.at[i2], bv, sem).wait()

        def _row(r, carry2):
            for cc in range(_EDD // 16):
                sl = pl.ds(cc * 16, 16)
                ev[r, sl] = 0.5 * (av[r, sl] + bv[r, sl])
            return carry2

        lax.fori_loop(0, _MC, _row, 0)
        pltpu.sync_copy(ev, es_hbm.at[pl.ds(base, _MC)])
        return carry

    lax.fori_loop(0, _MNCH, _ch, 0)


@functools.partial(
    pl.kernel,
    mesh=plsc.VectorSubcoreMesh(core_axis_name="c", subcore_axis_name="s"),
    out_type=jax.ShapeDtypeStruct((_E, _EDD), F32),
    scratch_types=[
        pltpu.VMEM((_MNCH, _MC), I32),
        pltpu.VMEM((_MNCH, _MC), I32),
        pltpu.VMEM((_MC,), I32),
        pltpu.VMEM((_MC,), I32),
        pltpu.VMEM((_MC,), I32),
        pltpu.VMEM((_MC,), I32),
        pltpu.VMEM((_MC,), I32),
        pltpu.VMEM((_MC,), I32),
        pltpu.VMEM((_MC, _IND), F32),
        pltpu.VMEM((_MC, _IND), F32),
        pltpu.VMEM((_MC, _EDD), F32),
        pltpu.SemaphoreType.DMA,
    ],
)
def _sc_head2(pm, ep, jr, ir, out,
              jv, iv, kb, rb, z1, z2, i1, i2, av, bv, ev, sem):
    _sc_head2_body(pm, ep, jr, ir, out,
                   jv, iv, kb, rb, z1, z2, i1, i2, av, bv, ev, sem)


def _node_upd_body(s_ref, p0_ref, p1_ref, wns_ref, wna_ref, bn_ref, so_ref):
    s = s_ref[...]
    agg = p0_ref[...] + p1_ref[...]
    h = _dot(s, wns_ref[...]) + _dot(agg, wna_ref[...]) + bn_ref[...]
    so_ref[...] = s + _silu(h)


def _node_update(s, p0, p1, wns, wna, bn):
    return pl.pallas_call(
        _node_upd_body,
        out_shape=jax.ShapeDtypeStruct((_N, _IND), F32),
    )(s, p0, p1, wns, wna, bn.reshape(1, -1))


def _head_node_body(s_ref, wsh_ref, bsh_ref, wal_ref, bal_ref,
                    s2_ref, lat_ref, at_ref):
    s2 = _silu(_dot(s_ref[...], wsh_ref[...]) + bsh_ref[...])
    ao = _dot(s2, wal_ref[...]) + bal_ref[...]
    s2_ref[...] = s2
    at_ref[...] = ao[:, :_NAF]
    lat_ref[...] = ao[:, _NAF:]


def _head_node(s, p):
    return pl.pallas_call(
        _head_node_body,
        out_shape=(jax.ShapeDtypeStruct((_N, _IND), F32),
                   jax.ShapeDtypeStruct((_N, _LAT), F32),
                   jax.ShapeDtypeStruct((_N, _NAF), F32)),
    )(s, p['h_sh_W'], p['h_sh_b'].reshape(1, -1),
      p['h_al_W'], p['h_al_b'].reshape(1, -1))


def _head_final_body(g_ref, es_ref, wbm_ref, bbm_ref, wbl_ref, bbl_ref,
                     bo_ref):
    f = _silu(g_ref[...] + _dot(es_ref[...], wbm_ref[...]) + bbm_ref[...])
    bo_ref[...] = _dot(f, wbl_ref[...]) + bbl_ref[...]


def _head_final(gsum, esym, p):
    nbt = p['h_bl_W'].shape[1]
    grid = (_E // ET,)
    return pl.pallas_call(
        _head_final_body,
        grid=grid,
        in_specs=[
            pl.BlockSpec((ET, _IND), lambda b: (b, 0)),
            pl.BlockSpec((ET, _EDD), lambda b: (b, 0)),
            pl.BlockSpec((_EDD, _IND), lambda b: (0, 0)),
            pl.BlockSpec((1, _IND), lambda b: (0, 0)),
            pl.BlockSpec((_IND, nbt), lambda b: (0, 0)),
            pl.BlockSpec((1, nbt), lambda b: (0, 0)),
        ],
        out_specs=pl.BlockSpec((ET, nbt), lambda b: (b, 0)),
        out_shape=jax.ShapeDtypeStruct((_E, nbt), F32),
    )(gsum, esym, p['h_bm_W'], p['h_bm_b'].reshape(1, -1),
      p['h_bl_W'], p['h_bl_b'].reshape(1, -1))


def kernel(x, t, z, edge_attr, params, edge_index, batch, batch_edge_global):
    p = params
    j = edge_index[0]
    i = edge_index[1]
    batch_col = batch.reshape(_N, 1)
    beg_col = batch_edge_global.reshape(_E, 1)
    jr3 = j.reshape(_NW, _MNCH, _MC)
    ir3 = i.reshape(_NW, _MNCH, _MC)

    s = _node_init(x, z, batch_col, t, p)
    e = _edge_init(edge_attr, beg_col, t, p)

    epad = None
    for l in range(_LGNN):
        wmsg = p['gnn_Wmsg'][l]
        wedge = p['gnn_Wedge'][l]
        wnode = p['gnn_Wnode'][l]
        pj, pi = _node_tables(s, wmsg[:_IND], wmsg[_IND:2 * _IND])
        q, r = _edge_pre(e, wmsg[2 * _IND:], p['gnn_bmsg'][l],
                         wedge[2 * _IND:], p['gnn_bedge'][l])
        parts = _sc_msg(pj, pi, q, jr3, ir3)
        s = _node_update(s, parts[0], parts[1],
                         wnode[:_IND], wnode[_IND:], p['gnn_bnode'][l])
        qt = _combo_table(s, wedge[:_IND], wedge[_IND:2 * _IND])
        if l < _LGNN - 1:
            e = _sc_edge(qt, e, r, jr3, ir3)
        else:
            epad = _sc_edge_pad(qt, e, r, jr3, ir3)

    s2, latent_pred, atoms_pred = _head_node(s, p)
    pmap, gsum = _sc_head1(s2, jr3, ir3)
    esym = _sc_head2(pmap, epad, jr3, ir3)
    bonds_pred = _head_final(gsum, esym, p)
    return latent_pred, atoms_pred, bonds_pred


# speedup vs baseline: 4.6358x; 1.3761x over previous
import functools

import jax
import jax.numpy as jnp
from jax import lax
from jax.experimental import pallas as pl
from jax.experimental.pallas import tpu as pltpu
from jax.experimental.pallas import tpu_sc as plsc

F32 = jnp.float32
I32 = jnp.int32
_N = 1024
_E = 131072
_B = 32
_IND = 128
_EDD = 32
_LGNN = 5
_NAF = 16
_LAT = 64
ET = 2048
_HIGH = jax.lax.Precision.DEFAULT


def _dot(a, b):
    return jnp.dot(a, b, precision=_HIGH, preferred_element_type=F32)


def _silu(v):
    return v * jax.nn.sigmoid(v)


def _onehot(idx_col, width):
    t = idx_col.shape[0]
    cols = lax.broadcasted_iota(I32, (t, width), 1)
    return jnp.where(idx_col == cols, 1.0, 0.0).astype(F32)


def _node_init_body(x_ref, z_ref, b_ref, t_ref, tmaW_ref, tmab_ref,
                    amW_ref, amb_ref, atmW_ref, atmb_ref, lmW_ref, lmb_ref,
                    s_ref):
    ta = _dot(t_ref[...], tmaW_ref[...]) + tmab_ref[...]
    t2 = _dot(ta, atmW_ref[...])
    a1 = _dot(amW_ref[...], atmW_ref[...])
    c2 = _dot(amb_ref[...], atmW_ref[...]) + atmb_ref[...] + lmb_ref[...]
    oh = _onehot(b_ref[...], _B)
    s_ref[...] = (_dot(x_ref[...], a1) + _dot(oh, t2)
                  + _dot(z_ref[...], lmW_ref[...]) + c2)


def _node_init(x, z, batch_col, t, p):
    return pl.pallas_call(
        _node_init_body,
        out_shape=jax.ShapeDtypeStruct((_N, _IND), F32),
    )(x, z, batch_col, t, p['tma_W'], p['tma_b'].reshape(1, -1),
      p['am_W'], p['am_b'].reshape(1, -1), p['atm_W'],
      p['atm_b'].reshape(1, -1), p['lm_W'], p['lm_b'].reshape(1, -1))


def _edge_init_body(ea_ref, beg_ref, t_ref, tmbW_ref, tmbb_ref,
                    bmW_ref, bmb_ref, btmW_ref, btmb_ref, e_ref):
    tb = _dot(t_ref[...], tmbW_ref[...]) + tmbb_ref[...]
    t1 = _dot(tb, btmW_ref[...])
    w1 = _dot(bmW_ref[...], btmW_ref[...])
    c1 = _dot(bmb_ref[...], btmW_ref[...]) + btmb_ref[...]
    oh = _onehot(beg_ref[...], _B)
    e_ref[...] = _dot(ea_ref[...], w1) + _dot(oh, t1) + c1


def _edge_init(edge_attr, beg_col, t, p):
    nbt = edge_attr.shape[1]
    grid = (_E // ET,)
    return pl.pallas_call(
        _edge_init_body,
        grid=grid,
        in_specs=[
            pl.BlockSpec((ET, nbt), lambda b: (b, 0)),
            pl.BlockSpec((ET, 1), lambda b: (b, 0)),
            pl.BlockSpec((_B, 1), lambda b: (0, 0)),
            pl.BlockSpec((1, _EDD), lambda b: (0, 0)),
            pl.BlockSpec((1, _EDD), lambda b: (0, 0)),
            pl.BlockSpec((nbt, _EDD), lambda b: (0, 0)),
            pl.BlockSpec((1, _EDD), lambda b: (0, 0)),
            pl.BlockSpec((_EDD, _EDD), lambda b: (0, 0)),
            pl.BlockSpec((1, _EDD), lambda b: (0, 0)),
        ],
        out_specs=pl.BlockSpec((ET, _EDD), lambda b: (b, 0)),
        out_shape=jax.ShapeDtypeStruct((_E, _EDD), F32),
    )(edge_attr, beg_col, t, p['tmb_W'], p['tmb_b'].reshape(1, -1),
      p['bm_W'], p['bm_b'].reshape(1, -1), p['btm_W'],
      p['btm_b'].reshape(1, -1))


def _tables_body(s_ref, wa_ref, wb_ref, a_ref, b_ref):
    a_ref[...] = _dot(s_ref[...], wa_ref[...])
    b_ref[...] = _dot(s_ref[...], wb_ref[...])


def _node_tables(s, wa, wb):
    return pl.pallas_call(
        _tables_body,
        out_shape=(jax.ShapeDtypeStruct((_N, wa.shape[1]), F32),
                   jax.ShapeDtypeStruct((_N, wb.shape[1]), F32)),
    )(s, wa, wb)


def _combo_body(s_ref, wa_ref, wb_ref, o_ref):
    s = s_ref[...]
    z = jnp.zeros((_N, _IND - 2 * _EDD), F32)
    o_ref[...] = jnp.concatenate(
        [_dot(s, wa_ref[...]), _dot(s, wb_ref[...]), z], axis=1)


def _combo_table(s, wa, wb):
    return pl.pallas_call(
        _combo_body,
        out_shape=jax.ShapeDtypeStruct((_N, _IND), F32),
    )(s, wa, wb)


def _edge_pre_body(e_ref, weq_ref, bq_ref, wer_ref, br_ref, q_ref, r_ref):
    e = e_ref[...]
    q_ref[...] = _dot(e, weq_ref[...]) + bq_ref[...]
    r_ref[...] = _dot(e, wer_ref[...]) + br_ref[...]


def _edge_pre(e, weq, bq, wer, br):
    grid = (_E // ET,)
    return pl.pallas_call(
        _edge_pre_body,
        grid=grid,
        in_specs=[
            pl.BlockSpec((ET, _EDD), lambda b: (b, 0)),
            pl.BlockSpec((_EDD, _IND), lambda b: (0, 0)),
            pl.BlockSpec((1, _IND), lambda b: (0, 0)),
            pl.BlockSpec((_EDD, _EDD), lambda b: (0, 0)),
            pl.BlockSpec((1, _EDD), lambda b: (0, 0)),
        ],
        out_specs=(pl.BlockSpec((ET, _IND), lambda b: (b, 0)),
                   pl.BlockSpec((ET, _EDD), lambda b: (b, 0))),
        out_shape=(jax.ShapeDtypeStruct((_E, _IND), F32),
                   jax.ShapeDtypeStruct((_E, _EDD), F32)),
    )(e, weq, bq.reshape(1, -1), wer, br.reshape(1, -1))


_NC = 2
_NS = 16
_NW = _NC * _NS
_EW = _E // _NW
_MC = 128
_MNCH = _EW // _MC
_EPAD = 2048


def _sc_msg_body(pj_hbm, pi_hbm, q_hbm, j_hbm, i_hbm, out_hbm,
                 jv, iv, av, bv, qv, zv, agg_sp, sem):
    c = lax.axis_index("c")
    s = lax.axis_index("s")
    w = s * _NC + c
    for cc in range(8):
        for ll in range(8):
            zv[cc, pl.ds(ll * 16, 16)] = jnp.zeros((16,), F32)
    rows_per = _N // _NS
    for k in range(rows_per // 8):
        pltpu.sync_copy(zv, agg_sp.at[pl.ds(s * rows_per + k * 8, 8)])
    pltpu.sync_copy(j_hbm.at[w], jv)
    pltpu.sync_copy(i_hbm.at[w], iv)
    plsc.subcore_barrier()
    for ch in range(_MNCH):
        base = w * _EW + ch * _MC
        pltpu.async_copy(pj_hbm.at[jv.at[ch]], av, sem).wait()
        pltpu.async_copy(pi_hbm.at[iv.at[ch]], bv, sem).wait()
        pltpu.sync_copy(q_hbm.at[pl.ds(base, _MC)], qv)

        def _row(r, carry):
            for cc in range(_IND // 16):
                sl = pl.ds(cc * 16, 16)
                v = av[r, sl] + bv[r, sl] + qv[r, sl]
                av[r, sl] = v / (1.0 + jnp.exp(-v))
            return carry

        lax.fori_loop(0, _MC, _row, 0)
        pltpu.sync_copy(av, agg_sp.at[iv.at[ch]], add=True)
    plsc.subcore_barrier()
    pltpu.sync_copy(agg_sp.at[pl.ds(s * rows_per, rows_per)],
                    out_hbm.at[c, pl.ds(s * rows_per, rows_per)])


@functools.partial(
    pl.kernel,
    mesh=plsc.VectorSubcoreMesh(core_axis_name="c", subcore_axis_name="s"),
    out_type=jax.ShapeDtypeStruct((_NC, _N, _IND), F32),
    scratch_types=[
        pltpu.VMEM((_MNCH, _MC), I32),
        pltpu.VMEM((_MNCH, _MC), I32),
        pltpu.VMEM((_MC, _IND), F32),
        pltpu.VMEM((_MC, _IND), F32),
        pltpu.VMEM((_MC, _IND), F32),
        pltpu.VMEM((8, _IND), F32),
        pltpu.VMEM_SHARED((_N, _IND), F32),
        pltpu.SemaphoreType.DMA,
    ],
)
def _sc_msg(pj, pi, q, jr, ir, out, jv, iv, av, bv, qv, zv, agg, sem):
    _sc_msg_body(pj, pi, q, jr, ir, out, jv, iv, av, bv, qv, zv, agg, sem)


def _sc_edge_common(qt_hbm, eo_hbm, r_hbm, j_hbm, i_hbm, out_hbm,
                    jv, iv, av, bv, ev, rv, sem, et=None):
    c = lax.axis_index("c")
    s = lax.axis_index("s")
    w = s * _NC + c
    pad = et is not None
    pltpu.sync_copy(j_hbm.at[w], jv)
    pltpu.sync_copy(i_hbm.at[w], iv)
    if pad:
        def _zr(r, carry):
            for cc in range(_IND // 16):
                ev[r, pl.ds(cc * 16, 16)] = jnp.zeros((16,), F32)
            return carry

        lax.fori_loop(0, _MC, _zr, 0)

        @pl.when(w == 0)
        def _pad_rows():
            def _pz(k, carry):
                pltpu.sync_copy(ev, out_hbm.at[pl.ds(_E + k * _MC, _MC)])
                return carry

            lax.fori_loop(0, _EPAD // _MC, _pz, 0)
    for ch in range(_MNCH):
        base = w * _EW + ch * _MC
        pltpu.async_copy(qt_hbm.at[jv.at[ch]], av, sem).wait()
        pltpu.async_copy(qt_hbm.at[iv.at[ch]], bv, sem).wait()
        pltpu.sync_copy(eo_hbm.at[pl.ds(base, _MC)], et if pad else ev)
        pltpu.sync_copy(r_hbm.at[pl.ds(base, _MC)], rv)

        def _row(rr, carry):
            for cc in range(_EDD // 16):
                sl = pl.ds(cc * 16, 16)
                v = (av[rr, sl] + bv[rr, pl.ds(_EDD + cc * 16, 16)]
                     + rv[rr, sl])
                eold = et[rr, sl] if pad else ev[rr, sl]
                ev[rr, sl] = eold + v / (1.0 + jnp.exp(-v))
            return carry

        lax.fori_loop(0, _MC, _row, 0)
        pltpu.sync_copy(ev, out_hbm.at[pl.ds(base, _MC)])


@functools.partial(
    pl.kernel,
    mesh=plsc.VectorSubcoreMesh(core_axis_name="c", subcore_axis_name="s"),
    out_type=jax.ShapeDtypeStruct((_E, _EDD), F32),
    scratch_types=[
        pltpu.VMEM((_MNCH, _MC), I32),
        pltpu.VMEM((_MNCH, _MC), I32),
        pltpu.VMEM((_MC, _IND), F32),
        pltpu.VMEM((_MC, _IND), F32),
        pltpu.VMEM((_MC, _EDD), F32),
        pltpu.VMEM((_MC, _EDD), F32),
        pltpu.SemaphoreType.DMA,
    ],
)
def _sc_edge(qt, e, r, jr, ir, out, jv, iv, av, bv, ev, rv, sem):
    _sc_edge_common(qt, e, r, jr, ir, out, jv, iv, av, bv, ev, rv, sem)


@functools.partial(
    pl.kernel,
    mesh=plsc.VectorSubcoreMesh(core_axis_name="c", subcore_axis_name="s"),
    out_type=jax.ShapeDtypeStruct((_E + _EPAD, _IND), F32),
    scratch_types=[
        pltpu.VMEM((_MNCH, _MC), I32),
        pltpu.VMEM((_MNCH, _MC), I32),
        pltpu.VMEM((_MC, _IND), F32),
        pltpu.VMEM((_MC, _IND), F32),
        pltpu.VMEM((_MC, _IND), F32),
        pltpu.VMEM((_MC, _EDD), F32),
        pltpu.VMEM((_MC, _EDD), F32),
        pltpu.SemaphoreType.DMA,
    ],
)
def _sc_edge_pad(qt, e, r, jr, ir, out, jv, iv, av, bv, ev, rv, et, sem):
    _sc_edge_common(qt, e, r, jr, ir, out, jv, iv, av, bv, ev, rv, sem,
                    et=et)


_PN = _N * _N
_PTOT = _PN + 16384
_ZW = _PTOT // _NS
_FIXR = 5


def _sc_head1_body(s2_hbm, j_hbm, i_hbm, p_hbm, g_hbm,
                   jv64, iv64, kx, vb, pv4, k2, zb, av, bv, sem, sem2):
    c = lax.axis_index("c")
    s = lax.axis_index("s")

    @pl.when(c == 0)
    def _build_p():
        pltpu.sync_copy(j_hbm.at[2 * s], jv64.at[pl.ds(0, 32)])
        pltpu.sync_copy(j_hbm.at[2 * s + 1], jv64.at[pl.ds(32, 32)])
        pltpu.sync_copy(i_hbm.at[2 * s], iv64.at[pl.ds(0, 32)])
        pltpu.sync_copy(i_hbm.at[2 * s + 1], iv64.at[pl.ds(32, 32)])
        for g in range(64):
            zb[pl.ds(g * 16, 16)] = jnp.zeros((16,), I32)

        def _z(k, carry):
            pltpu.sync_copy(zb, p_hbm.at[pl.ds(s * _ZW + k * 1024, 1024)])
            return carry

        lax.fori_loop(0, _ZW // 1024, _z, 0)

        def _kv(ch, carry):
            for g in range(8):
                sl = pl.ds(g * 16, 16)
                kx[ch, sl] = jv64[ch, sl] * _N + iv64[ch, sl]
                vb[ch, sl] = (s * 8192 + ch * 128 + g * 16 + 1
                              + lax.iota(I32, 16))
            return carry

        lax.fori_loop(0, 64, _kv, 0)
        plsc.subcore_barrier()

        def _r1(c2, carry):
            hs = [pltpu.async_copy(vb.at[c2 * 4 + u], p_hbm.at[kx.at[c2 * 4 + u]],
                                   sem) for u in range(4)]
            for h in hs:
                h.wait()
            return carry

        lax.fori_loop(0, 16, _r1, 0)
        plsc.subcore_barrier()
        for _ in range(_FIXR):
            def _fr(c2, carry):
                hs = [pltpu.async_copy(p_hbm.at[kx.at[c2 * 4 + u]], pv4.at[u],
                                       sem) for u in range(4)]
                for h in hs:
                    h.wait()
                for u in range(4):
                    ch = c2 * 4 + u
                    for g in range(8):
                        sl = pl.ds(g * 16, 16)
                        mywin = vb[ch, sl] > pv4[u, sl]
                        dump = (_PN + s * 1024
                                + ((ch * 128 + g * 16) % 1024)
                                + lax.iota(I32, 16))
                        k2[u, sl] = jnp.where(mywin, kx[ch, sl], dump)
                hs2 = [pltpu.async_copy(vb.at[c2 * 4 + u], p_hbm.at[k2.at[u]],
                                        sem) for u in range(4)]
                for h in hs2:
                    h.wait()
                return carry

            lax.fori_loop(0, 16, _fr, 0)
            plsc.subcore_barrier()

    @pl.when(c == 1)
    def _gsum():
        pltpu.sync_copy(j_hbm.at[2 * s], jv64.at[pl.ds(0, 32)])
        pltpu.sync_copy(j_hbm.at[2 * s + 1], jv64.at[pl.ds(32, 32)])
        pltpu.sync_copy(i_hbm.at[2 * s], iv64.at[pl.ds(0, 32)])
        pltpu.sync_copy(i_hbm.at[2 * s + 1], iv64.at[pl.ds(32, 32)])

        def _gs(ch, carry):
            pltpu.async_copy(s2_hbm.at[jv64.at[ch]], av, sem2).wait()
            pltpu.async_copy(s2_hbm.at[iv64.at[ch]], bv, sem2).wait()

            def _row(r, carry2):
                for g in range(8):
                    sl = pl.ds(g * 16, 16)
                    av[r, sl] = av[r, sl] + bv[r, sl]
                return carry2

            lax.fori_loop(0, 128, _row, 0)
            pltpu.sync_copy(av, g_hbm.at[pl.ds(s * 8192 + ch * 128, 128)])
            return carry

        lax.fori_loop(0, 64, _gs, 0)


@functools.partial(
    pl.kernel,
    mesh=plsc.VectorSubcoreMesh(core_axis_name="c", subcore_axis_name="s"),
    out_type=(jax.ShapeDtypeStruct((_PTOT,), I32),
              jax.ShapeDtypeStruct((_E, _IND), F32)),
    scratch_types=[
        pltpu.VMEM((64, 128), I32),
        pltpu.VMEM((64, 128), I32),
        pltpu.VMEM((64, 128), I32),
        pltpu.VMEM((64, 128), I32),
        pltpu.VMEM((4, 128), I32),
        pltpu.VMEM((4, 128), I32),
        pltpu.VMEM((1024,), I32),
        pltpu.VMEM((128, 128), F32),
        pltpu.VMEM((128, 128), F32),
        pltpu.SemaphoreType.DMA,
        pltpu.SemaphoreType.DMA,
    ],
)
def _sc_head1(s2, jr, ir, p_out, g_out,
              jv64, iv64, kx, vb, pv4, k2, zb, av, bv, sem, sem2):
    _sc_head1_body(s2, jr, ir, p_out, g_out,
                   jv64, iv64, kx, vb, pv4, k2, zb, av, bv, sem, sem2)


def _sc_head2_body(p_hbm, ep_hbm, j_hbm, i_hbm, es_hbm,
                   jv, iv, kb, rb, z1, z2, i1, i2, av, bv, ev, sem):
    c = lax.axis_index("c")
    s = lax.axis_index("s")
    w = s * _NC + c
    pltpu.sync_copy(j_hbm.at[w], jv)
    pltpu.sync_copy(i_hbm.at[w], iv)

    def _ch(ch, carry):
        base = w * _EW + ch * _MC
        for g in range(8):
            sl = pl.ds(g * 16, 16)
            kb[sl] = jv[ch, sl] * _N + iv[ch, sl]
            rb[sl] = iv[ch, sl] * _N + jv[ch, sl]
        pltpu.async_copy(p_hbm.at[kb], z1, sem).wait()
        pltpu.async_copy(p_hbm.at[rb], z2, sem).wait()
        for g in range(8):
            sl = pl.ds(g * 16, 16)
            i1[sl] = z1[sl] - 1
            dummy = _E + ((kb[sl] + g * 16 + lax.iota(I32, 16)) & (_EPAD - 1))
            i2[sl] = jnp.where(z2[sl] > 0, z2[sl] - 1, dummy)
        pltpu.async_copy(ep_hbm.at[i1], av, sem).wait()
        pltpu.async_copy(ep_hbm.at[i2], bv, sem).wait()

        def _row(r, carry2):
            for cc in range(_EDD // 16):
                sl = pl.ds(cc * 16, 16)
                ev[r, sl] = 0.5 * (av[r, sl] + bv[r, sl])
            return carry2

        lax.fori_loop(0, _MC, _row, 0)
        pltpu.sync_copy(ev, es_hbm.at[pl.ds(base, _MC)])
        return carry

    lax.fori_loop(0, _MNCH, _ch, 0)


@functools.partial(
    pl.kernel,
    mesh=plsc.VectorSubcoreMesh(core_axis_name="c", subcore_axis_name="s"),
    out_type=jax.ShapeDtypeStruct((_E, _EDD), F32),
    scratch_types=[
        pltpu.VMEM((_MNCH, _MC), I32),
        pltpu.VMEM((_MNCH, _MC), I32),
        pltpu.VMEM((_MC,), I32),
        pltpu.VMEM((_MC,), I32),
        pltpu.VMEM((_MC,), I32),
        pltpu.VMEM((_MC,), I32),
        pltpu.VMEM((_MC,), I32),
        pltpu.VMEM((_MC,), I32),
        pltpu.VMEM((_MC, _IND), F32),
        pltpu.VMEM((_MC, _IND), F32),
        pltpu.VMEM((_MC, _EDD), F32),
        pltpu.SemaphoreType.DMA,
    ],
)
def _sc_head2(pm, ep, jr, ir, out,
              jv, iv, kb, rb, z1, z2, i1, i2, av, bv, ev, sem):
    _sc_head2_body(pm, ep, jr, ir, out,
                   jv, iv, kb, rb, z1, z2, i1, i2, av, bv, ev, sem)


def _node_upd_body(s_ref, p0_ref, p1_ref, wns_ref, wna_ref, bn_ref, so_ref):
    s = s_ref[...]
    agg = p0_ref[...] + p1_ref[...]
    h = _dot(s, wns_ref[...]) + _dot(agg, wna_ref[...]) + bn_ref[...]
    so_ref[...] = s + _silu(h)


def _node_update(s, p0, p1, wns, wna, bn):
    return pl.pallas_call(
        _node_upd_body,
        out_shape=jax.ShapeDtypeStruct((_N, _IND), F32),
    )(s, p0, p1, wns, wna, bn.reshape(1, -1))


def _head_node_body(s_ref, wsh_ref, bsh_ref, wal_ref, bal_ref,
                    s2_ref, lat_ref, at_ref):
    s2 = _silu(_dot(s_ref[...], wsh_ref[...]) + bsh_ref[...])
    ao = _dot(s2, wal_ref[...]) + bal_ref[...]
    s2_ref[...] = s2
    at_ref[...] = ao[:, :_NAF]
    lat_ref[...] = ao[:, _NAF:]


def _head_node(s, p):
    return pl.pallas_call(
        _head_node_body,
        out_shape=(jax.ShapeDtypeStruct((_N, _IND), F32),
                   jax.ShapeDtypeStruct((_N, _LAT), F32),
                   jax.ShapeDtypeStruct((_N, _NAF), F32)),
    )(s, p['h_sh_W'], p['h_sh_b'].reshape(1, -1),
      p['h_al_W'], p['h_al_b'].reshape(1, -1))


def _head_final_body(g_ref, es_ref, wbm_ref, bbm_ref, wbl_ref, bbl_ref,
                     bo_ref):
    f = _silu(g_ref[...] + _dot(es_ref[...], wbm_ref[...]) + bbm_ref[...])
    bo_ref[...] = _dot(f, wbl_ref[...]) + bbl_ref[...]


def _head_final(gsum, esym, p):
    nbt = p['h_bl_W'].shape[1]
    grid = (_E // ET,)
    return pl.pallas_call(
        _head_final_body,
        grid=grid,
        in_specs=[
            pl.BlockSpec((ET, _IND), lambda b: (b, 0)),
            pl.BlockSpec((ET, _EDD), lambda b: (b, 0)),
            pl.BlockSpec((_EDD, _IND), lambda b: (0, 0)),
            pl.BlockSpec((1, _IND), lambda b: (0, 0)),
            pl.BlockSpec((_IND, nbt), lambda b: (0, 0)),
            pl.BlockSpec((1, nbt), lambda b: (0, 0)),
        ],
        out_specs=pl.BlockSpec((ET, nbt), lambda b: (b, 0)),
        out_shape=jax.ShapeDtypeStruct((_E, nbt), F32),
    )(gsum, esym, p['h_bm_W'], p['h_bm_b'].reshape(1, -1),
      p['h_bl_W'], p['h_bl_b'].reshape(1, -1))


def kernel(x, t, z, edge_attr, params, edge_index, batch, batch_edge_global):
    p = params
    j = edge_index[0]
    i = edge_index[1]
    batch_col = batch.reshape(_N, 1)
    beg_col = batch_edge_global.reshape(_E, 1)
    jr3 = j.reshape(_NW, _MNCH, _MC)
    ir3 = i.reshape(_NW, _MNCH, _MC)

    s = _node_init(x, z, batch_col, t, p)
    e = _edge_init(edge_attr, beg_col, t, p)

    epad = None
    for l in range(_LGNN):
        wmsg = p['gnn_Wmsg'][l]
        wedge = p['gnn_Wedge'][l]
        wnode = p['gnn_Wnode'][l]
        pj, pi = _node_tables(s, wmsg[:_IND], wmsg[_IND:2 * _IND])
        q, r = _edge_pre(e, wmsg[2 * _IND:], p['gnn_bmsg'][l],
                         wedge[2 * _IND:], p['gnn_bedge'][l])
        parts = _sc_msg(pj, pi, q, jr3, ir3)
        s = _node_update(s, parts[0], parts[1],
                         wnode[:_IND], wnode[_IND:], p['gnn_bnode'][l])
        qt = _combo_table(s, wedge[:_IND], wedge[_IND:2 * _IND])
        if l < _LGNN - 1:
            e = _sc_edge(qt, e, r, jr3, ir3)
        else:
            epad = _sc_edge_pad(qt, e, r, jr3, ir3)

    s2, latent_pred, atoms_pred = _head_node(s, p)
    pmap, gsum = _sc_head1(s2, jr3, ir3)
    esym = _sc_head2(pmap, epad, jr3, ir3)
    bonds_pred = _head_final(gsum, esym, p)
    return latent_pred, atoms_pred, bonds_pred


# speedup vs baseline: 5.2817x; 1.1393x over previous
import functools

import jax
import jax.numpy as jnp
from jax import lax
from jax.experimental import pallas as pl
from jax.experimental.pallas import tpu as pltpu
from jax.experimental.pallas import tpu_sc as plsc

F32 = jnp.float32
I32 = jnp.int32
_N = 1024
_E = 131072
_B = 32
_IND = 128
_EDD = 32
_LGNN = 5
_NAF = 16
_LAT = 64
ET = 2048
_HIGH = jax.lax.Precision.DEFAULT


def _dot(a, b):
    return jnp.dot(a, b, precision=_HIGH, preferred_element_type=F32)


def _silu(v):
    return v * jax.nn.sigmoid(v)


def _onehot(idx_col, width):
    t = idx_col.shape[0]
    cols = lax.broadcasted_iota(I32, (t, width), 1)
    return jnp.where(idx_col == cols, 1.0, 0.0).astype(F32)


def _node_init_body(x_ref, z_ref, b_ref, t_ref, tmaW_ref, tmab_ref,
                    amW_ref, amb_ref, atmW_ref, atmb_ref, lmW_ref, lmb_ref,
                    s_ref):
    ta = _dot(t_ref[...], tmaW_ref[...]) + tmab_ref[...]
    t2 = _dot(ta, atmW_ref[...])
    a1 = _dot(amW_ref[...], atmW_ref[...])
    c2 = _dot(amb_ref[...], atmW_ref[...]) + atmb_ref[...] + lmb_ref[...]
    oh = _onehot(b_ref[...], _B)
    s_ref[...] = (_dot(x_ref[...], a1) + _dot(oh, t2)
                  + _dot(z_ref[...], lmW_ref[...]) + c2)


def _node_init(x, z, batch_col, t, p):
    return pl.pallas_call(
        _node_init_body,
        out_shape=jax.ShapeDtypeStruct((_N, _IND), F32),
    )(x, z, batch_col, t, p['tma_W'], p['tma_b'].reshape(1, -1),
      p['am_W'], p['am_b'].reshape(1, -1), p['atm_W'],
      p['atm_b'].reshape(1, -1), p['lm_W'], p['lm_b'].reshape(1, -1))


def _edge_init_body(ea_ref, beg_ref, t_ref, tmbW_ref, tmbb_ref,
                    bmW_ref, bmb_ref, btmW_ref, btmb_ref, e_ref):
    tb = _dot(t_ref[...], tmbW_ref[...]) + tmbb_ref[...]
    t1 = _dot(tb, btmW_ref[...])
    w1 = _dot(bmW_ref[...], btmW_ref[...])
    c1 = _dot(bmb_ref[...], btmW_ref[...]) + btmb_ref[...]
    oh = _onehot(beg_ref[...], _B)
    e_ref[...] = _dot(ea_ref[...], w1) + _dot(oh, t1) + c1


def _edge_init(edge_attr, beg_col, t, p):
    nbt = edge_attr.shape[1]
    grid = (_E // ET,)
    return pl.pallas_call(
        _edge_init_body,
        grid=grid,
        in_specs=[
            pl.BlockSpec((ET, nbt), lambda b: (b, 0)),
            pl.BlockSpec((ET, 1), lambda b: (b, 0)),
            pl.BlockSpec((_B, 1), lambda b: (0, 0)),
            pl.BlockSpec((1, _EDD), lambda b: (0, 0)),
            pl.BlockSpec((1, _EDD), lambda b: (0, 0)),
            pl.BlockSpec((nbt, _EDD), lambda b: (0, 0)),
            pl.BlockSpec((1, _EDD), lambda b: (0, 0)),
            pl.BlockSpec((_EDD, _EDD), lambda b: (0, 0)),
            pl.BlockSpec((1, _EDD), lambda b: (0, 0)),
        ],
        out_specs=pl.BlockSpec((ET, _EDD), lambda b: (b, 0)),
        out_shape=jax.ShapeDtypeStruct((_E, _EDD), F32),
    )(edge_attr, beg_col, t, p['tmb_W'], p['tmb_b'].reshape(1, -1),
      p['bm_W'], p['bm_b'].reshape(1, -1), p['btm_W'],
      p['btm_b'].reshape(1, -1))


def _tables_body(s_ref, wa_ref, wb_ref, a_ref, b_ref):
    a_ref[...] = _dot(s_ref[...], wa_ref[...])
    b_ref[...] = _dot(s_ref[...], wb_ref[...])


def _node_tables(s, wa, wb):
    return pl.pallas_call(
        _tables_body,
        out_shape=(jax.ShapeDtypeStruct((_N, wa.shape[1]), F32),
                   jax.ShapeDtypeStruct((_N, wb.shape[1]), F32)),
    )(s, wa, wb)


def _combo_body(s_ref, wa_ref, wb_ref, o_ref):
    s = s_ref[...]
    z = jnp.zeros((_N, _IND - 2 * _EDD), F32)
    o_ref[...] = jnp.concatenate(
        [_dot(s, wa_ref[...]), _dot(s, wb_ref[...]), z], axis=1)


def _combo_table(s, wa, wb):
    return pl.pallas_call(
        _combo_body,
        out_shape=jax.ShapeDtypeStruct((_N, _IND), F32),
    )(s, wa, wb)


def _edge_pre_body(e_ref, weq_ref, bq_ref, wer_ref, br_ref, q_ref, r_ref):
    e = e_ref[...]
    q_ref[...] = _dot(e, weq_ref[...]) + bq_ref[...]
    r_ref[...] = _dot(e, wer_ref[...]) + br_ref[...]


def _edge_pre(e, weq, bq, wer, br):
    grid = (_E // ET,)
    return pl.pallas_call(
        _edge_pre_body,
        grid=grid,
        in_specs=[
            pl.BlockSpec((ET, _EDD), lambda b: (b, 0)),
            pl.BlockSpec((_EDD, _IND), lambda b: (0, 0)),
            pl.BlockSpec((1, _IND), lambda b: (0, 0)),
            pl.BlockSpec((_EDD, _EDD), lambda b: (0, 0)),
            pl.BlockSpec((1, _EDD), lambda b: (0, 0)),
        ],
        out_specs=(pl.BlockSpec((ET, _IND), lambda b: (b, 0)),
                   pl.BlockSpec((ET, _EDD), lambda b: (b, 0))),
        out_shape=(jax.ShapeDtypeStruct((_E, _IND), F32),
                   jax.ShapeDtypeStruct((_E, _EDD), F32)),
    )(e, weq, bq.reshape(1, -1), wer, br.reshape(1, -1))


_NC = 2
_NS = 16
_NW = _NC * _NS
_EW = _E // _NW
_MC = 128
_MNCH = _EW // _MC
_EPAD = 2048


_MCM = 64
_MNCHM = _EW // _MCM


def _sc_msg_body(pj_hbm, pi_hbm, q_hbm, j_hbm, i_hbm, out_hbm,
                 jv, iv, av0, bv0, av1, bv1, qv, zv, agg_sp, s0, s1):
    c = lax.axis_index("c")
    s = lax.axis_index("s")
    w = s * _NC + c
    for cc in range(8):
        for ll in range(8):
            zv[cc, pl.ds(ll * 16, 16)] = jnp.zeros((16,), F32)
    rows_per = _N // _NS
    for k in range(rows_per // 8):
        pltpu.sync_copy(zv, agg_sp.at[pl.ds(s * rows_per + k * 8, 8)])
    pltpu.sync_copy(j_hbm.at[w], jv)
    pltpu.sync_copy(i_hbm.at[w], iv)
    plsc.subcore_barrier()
    bufs = [(av0, bv0, s0), (av1, bv1, s1)]

    def _fire(ch, slot):
        av, bv, sm = slot
        pltpu.async_copy(pj_hbm.at[jv.at[ch]], av, sm)
        pltpu.async_copy(pi_hbm.at[iv.at[ch]], bv, sm)

    def _consume(ch, slot):
        av, bv, sm = slot
        base = w * _EW + ch * _MCM
        pltpu.sync_copy(q_hbm.at[pl.ds(base, _MCM)], qv)
        pltpu.make_async_copy(pj_hbm.at[jv.at[0]], av, sm).wait()
        pltpu.make_async_copy(pi_hbm.at[iv.at[0]], bv, sm).wait()

        def _row(r, carry):
            for cc in range(_IND // 16):
                sl = pl.ds(cc * 16, 16)
                v = av[r, sl] + bv[r, sl] + qv[r, sl]
                av[r, sl] = v / (1.0 + jnp.exp(-v))
            return carry

        lax.fori_loop(0, _MCM, _row, 0)
        pltpu.sync_copy(av, agg_sp.at[iv.at[ch]], add=True)

    nit = _MNCHM // 2
    _fire(0, bufs[0])

    def _pair(it, carry):
        ch0 = it * 2
        _fire(ch0 + 1, bufs[1])
        _consume(ch0, bufs[0])

        @pl.when(it + 1 < nit)
        def _pref():
            _fire(ch0 + 2, bufs[0])

        _consume(ch0 + 1, bufs[1])
        return carry

    lax.fori_loop(0, nit, _pair, 0)
    plsc.subcore_barrier()
    pltpu.sync_copy(agg_sp.at[pl.ds(s * rows_per, rows_per)],
                    out_hbm.at[c, pl.ds(s * rows_per, rows_per)])


@functools.partial(
    pl.kernel,
    mesh=plsc.VectorSubcoreMesh(core_axis_name="c", subcore_axis_name="s"),
    out_type=jax.ShapeDtypeStruct((_NC, _N, _IND), F32),
    scratch_types=[
        pltpu.VMEM((_MNCHM, _MCM), I32),
        pltpu.VMEM((_MNCHM, _MCM), I32),
        pltpu.VMEM((_MCM, _IND), F32),
        pltpu.VMEM((_MCM, _IND), F32),
        pltpu.VMEM((_MCM, _IND), F32),
        pltpu.VMEM((_MCM, _IND), F32),
        pltpu.VMEM((_MCM, _IND), F32),
        pltpu.VMEM((8, _IND), F32),
        pltpu.VMEM_SHARED((_N, _IND), F32),
        pltpu.SemaphoreType.DMA,
        pltpu.SemaphoreType.DMA,
    ],
)
def _sc_msg(pj, pi, q, jr, ir, out, jv, iv, av0, bv0, av1, bv1, qv,
            zv, agg, s0, s1):
    _sc_msg_body(pj, pi, q, jr, ir, out, jv, iv, av0, bv0,
                 av1, bv1, qv, zv, agg, s0, s1)


def _sc_edge_common(qt_hbm, eo_hbm, r_hbm, j_hbm, i_hbm, out_hbm,
                    jv, iv, av, bv, ev, rv, sem, et=None):
    c = lax.axis_index("c")
    s = lax.axis_index("s")
    w = s * _NC + c
    pad = et is not None
    pltpu.sync_copy(j_hbm.at[w], jv)
    pltpu.sync_copy(i_hbm.at[w], iv)
    if pad:
        def _zr(r, carry):
            for cc in range(_IND // 16):
                ev[r, pl.ds(cc * 16, 16)] = jnp.zeros((16,), F32)
            return carry

        lax.fori_loop(0, _MC, _zr, 0)

        @pl.when(w == 0)
        def _pad_rows():
            def _pz(k, carry):
                pltpu.sync_copy(ev, out_hbm.at[pl.ds(_E + k * _MC, _MC)])
                return carry

            lax.fori_loop(0, _EPAD // _MC, _pz, 0)
    for ch in range(_MNCH):
        base = w * _EW + ch * _MC
        pltpu.async_copy(qt_hbm.at[jv.at[ch]], av, sem).wait()
        pltpu.async_copy(qt_hbm.at[iv.at[ch]], bv, sem).wait()
        pltpu.sync_copy(eo_hbm.at[pl.ds(base, _MC)], et if pad else ev)
        pltpu.sync_copy(r_hbm.at[pl.ds(base, _MC)], rv)

        def _row(rr, carry):
            for cc in range(_EDD // 16):
                sl = pl.ds(cc * 16, 16)
                v = (av[rr, sl] + bv[rr, pl.ds(_EDD + cc * 16, 16)]
                     + rv[rr, sl])
                eold = et[rr, sl] if pad else ev[rr, sl]
                ev[rr, sl] = eold + v / (1.0 + jnp.exp(-v))
            return carry

        lax.fori_loop(0, _MC, _row, 0)
        pltpu.sync_copy(ev, out_hbm.at[pl.ds(base, _MC)])


@functools.partial(
    pl.kernel,
    mesh=plsc.VectorSubcoreMesh(core_axis_name="c", subcore_axis_name="s"),
    out_type=jax.ShapeDtypeStruct((_E, _EDD), F32),
    scratch_types=[
        pltpu.VMEM((_MNCH, _MC), I32),
        pltpu.VMEM((_MNCH, _MC), I32),
        pltpu.VMEM((_MC, _IND), F32),
        pltpu.VMEM((_MC, _IND), F32),
        pltpu.VMEM((_MC, _EDD), F32),
        pltpu.VMEM((_MC, _EDD), F32),
        pltpu.SemaphoreType.DMA,
    ],
)
def _sc_edge(qt, e, r, jr, ir, out, jv, iv, av, bv, ev, rv, sem):
    _sc_edge_common(qt, e, r, jr, ir, out, jv, iv, av, bv, ev, rv, sem)


@functools.partial(
    pl.kernel,
    mesh=plsc.VectorSubcoreMesh(core_axis_name="c", subcore_axis_name="s"),
    out_type=jax.ShapeDtypeStruct((_E + _EPAD, _IND), F32),
    scratch_types=[
        pltpu.VMEM((_MNCH, _MC), I32),
        pltpu.VMEM((_MNCH, _MC), I32),
        pltpu.VMEM((_MC, _IND), F32),
        pltpu.VMEM((_MC, _IND), F32),
        pltpu.VMEM((_MC, _IND), F32),
        pltpu.VMEM((_MC, _EDD), F32),
        pltpu.VMEM((_MC, _EDD), F32),
        pltpu.SemaphoreType.DMA,
    ],
)
def _sc_edge_pad(qt, e, r, jr, ir, out, jv, iv, av, bv, ev, rv, et, sem):
    _sc_edge_common(qt, e, r, jr, ir, out, jv, iv, av, bv, ev, rv, sem,
                    et=et)


_PN = _N * _N
_PTOT = _PN + 16384
_ZW = _PTOT // _NS
_FIXR = 5


def _sc_head1_body(s2_hbm, j_hbm, i_hbm, p_hbm, g_hbm,
                   jv64, iv64, kx, vb, pv4, k2, zb, av, bv, sem, sem2):
    c = lax.axis_index("c")
    s = lax.axis_index("s")

    @pl.when(c == 0)
    def _build_p():
        pltpu.sync_copy(j_hbm.at[2 * s], jv64.at[pl.ds(0, 32)])
        pltpu.sync_copy(j_hbm.at[2 * s + 1], jv64.at[pl.ds(32, 32)])
        pltpu.sync_copy(i_hbm.at[2 * s], iv64.at[pl.ds(0, 32)])
        pltpu.sync_copy(i_hbm.at[2 * s + 1], iv64.at[pl.ds(32, 32)])
        for g in range(64):
            zb[pl.ds(g * 16, 16)] = jnp.zeros((16,), I32)

        def _z(k, carry):
            pltpu.sync_copy(zb, p_hbm.at[pl.ds(s * _ZW + k * 1024, 1024)])
            return carry

        lax.fori_loop(0, _ZW // 1024, _z, 0)

        def _kv(ch, carry):
            for g in range(8):
                sl = pl.ds(g * 16, 16)
                kx[ch, sl] = jv64[ch, sl] * _N + iv64[ch, sl]
                vb[ch, sl] = (s * 8192 + ch * 128 + g * 16 + 1
                              + lax.iota(I32, 16))
            return carry

        lax.fori_loop(0, 64, _kv, 0)
        plsc.subcore_barrier()

        def _r1(c2, carry):
            hs = [pltpu.async_copy(vb.at[c2 * 4 + u], p_hbm.at[kx.at[c2 * 4 + u]],
                                   sem) for u in range(4)]
            for h in hs:
                h.wait()
            return carry

        lax.fori_loop(0, 16, _r1, 0)
        plsc.subcore_barrier()
        for _ in range(_FIXR):
            def _fr(c2, carry):
                hs = [pltpu.async_copy(p_hbm.at[kx.at[c2 * 4 + u]], pv4.at[u],
                                       sem) for u in range(4)]
                for h in hs:
                    h.wait()
                for u in range(4):
                    ch = c2 * 4 + u
                    for g in range(8):
                        sl = pl.ds(g * 16, 16)
                        mywin = vb[ch, sl] > pv4[u, sl]
                        dump = (_PN + s * 1024
                                + ((ch * 128 + g * 16) % 1024)
                                + lax.iota(I32, 16))
                        k2[u, sl] = jnp.where(mywin, kx[ch, sl], dump)
                hs2 = [pltpu.async_copy(vb.at[c2 * 4 + u], p_hbm.at[k2.at[u]],
                                        sem) for u in range(4)]
                for h in hs2:
                    h.wait()
                return carry

            lax.fori_loop(0, 16, _fr, 0)
            plsc.subcore_barrier()

    @pl.when(c == 1)
    def _gsum():
        pltpu.sync_copy(j_hbm.at[2 * s], jv64.at[pl.ds(0, 32)])
        pltpu.sync_copy(j_hbm.at[2 * s + 1], jv64.at[pl.ds(32, 32)])
        pltpu.sync_copy(i_hbm.at[2 * s], iv64.at[pl.ds(0, 32)])
        pltpu.sync_copy(i_hbm.at[2 * s + 1], iv64.at[pl.ds(32, 32)])

        def _gs(ch, carry):
            pltpu.async_copy(s2_hbm.at[jv64.at[ch]], av, sem2).wait()
            pltpu.async_copy(s2_hbm.at[iv64.at[ch]], bv, sem2).wait()

            def _row(r, carry2):
                for g in range(8):
                    sl = pl.ds(g * 16, 16)
                    av[r, sl] = av[r, sl] + bv[r, sl]
                return carry2

            lax.fori_loop(0, 128, _row, 0)
            pltpu.sync_copy(av, g_hbm.at[pl.ds(s * 8192 + ch * 128, 128)])
            return carry

        lax.fori_loop(0, 64, _gs, 0)


@functools.partial(
    pl.kernel,
    mesh=plsc.VectorSubcoreMesh(core_axis_name="c", subcore_axis_name="s"),
    out_type=(jax.ShapeDtypeStruct((_PTOT,), I32),
              jax.ShapeDtypeStruct((_E, _IND), F32)),
    scratch_types=[
        pltpu.VMEM((64, 128), I32),
        pltpu.VMEM((64, 128), I32),
        pltpu.VMEM((64, 128), I32),
        pltpu.VMEM((64, 128), I32),
        pltpu.VMEM((4, 128), I32),
        pltpu.VMEM((4, 128), I32),
        pltpu.VMEM((1024,), I32),
        pltpu.VMEM((128, 128), F32),
        pltpu.VMEM((128, 128), F32),
        pltpu.SemaphoreType.DMA,
        pltpu.SemaphoreType.DMA,
    ],
)
def _sc_head1(s2, jr, ir, p_out, g_out,
              jv64, iv64, kx, vb, pv4, k2, zb, av, bv, sem, sem2):
    _sc_head1_body(s2, jr, ir, p_out, g_out,
                   jv64, iv64, kx, vb, pv4, k2, zb, av, bv, sem, sem2)


def _sc_head2_body(p_hbm, ep_hbm, j_hbm, i_hbm, es_hbm,
                   jv, iv, kb, rb, z1, z2, i1, i2, av, bv, ev, sem):
    c = lax.axis_index("c")
    s = lax.axis_index("s")
    w = s * _NC + c
    pltpu.sync_copy(j_hbm.at[w], jv)
    pltpu.sync_copy(i_hbm.at[w], iv)

    def _ch(ch, carry):
        base = w * _EW + ch * _MC
        for g in range(8):
            sl = pl.ds(g * 16, 16)
            kb[sl] = jv[ch, sl] * _N + iv[ch, sl]
            rb[sl] = iv[ch, sl] * _N + jv[ch, sl]
        pltpu.async_copy(p_hbm.at[kb], z1, sem).wait()
        pltpu.async_copy(p_hbm.at[rb], z2, sem).wait()
        for g in range(8):
            sl = pl.ds(g * 16, 16)
            i1[sl] = z1[sl] - 1
            dummy = _E + ((kb[sl] + g * 16 + lax.iota(I32, 16)) & (_EPAD - 1))
            i2[sl] = jnp.where(z2[sl] > 0, z2[sl] - 1, dummy)
        pltpu.async_copy(ep_hbm.at[i1], av, sem).wait()
        pltpu.async_copy(ep_hbm.at[i2], bv, sem).wait()

        def _row(r, carry2):
            for cc in range(_EDD // 16):
                sl = pl.ds(cc * 16, 16)
                ev[r, sl] = 0.5 * (av[r, sl] + bv[r, sl])
            return carry2

        lax.fori_loop(0, _MC, _row, 0)
        pltpu.sync_copy(ev, es_hbm.at[pl.ds(base, _MC)])
        return carry

    lax.fori_loop(0, _MNCH, _ch, 0)


@functools.partial(
    pl.kernel,
    mesh=plsc.VectorSubcoreMesh(core_axis_name="c", subcore_axis_name="s"),
    out_type=jax.ShapeDtypeStruct((_E, _EDD), F32),
    scratch_types=[
        pltpu.VMEM((_MNCH, _MC), I32),
        pltpu.VMEM((_MNCH, _MC), I32),
        pltpu.VMEM((_MC,), I32),
        pltpu.VMEM((_MC,), I32),
        pltpu.VMEM((_MC,), I32),
        pltpu.VMEM((_MC,), I32),
        pltpu.VMEM((_MC,), I32),
        pltpu.VMEM((_MC,), I32),
        pltpu.VMEM((_MC, _IND), F32),
        pltpu.VMEM((_MC, _IND), F32),
        pltpu.VMEM((_MC, _EDD), F32),
        pltpu.SemaphoreType.DMA,
    ],
)
def _sc_head2(pm, ep, jr, ir, out,
              jv, iv, kb, rb, z1, z2, i1, i2, av, bv, ev, sem):
    _sc_head2_body(pm, ep, jr, ir, out,
                   jv, iv, kb, rb, z1, z2, i1, i2, av, bv, ev, sem)


def _node_upd_body(s_ref, p0_ref, p1_ref, wns_ref, wna_ref, bn_ref, so_ref):
    s = s_ref[...]
    agg = p0_ref[...] + p1_ref[...]
    h = _dot(s, wns_ref[...]) + _dot(agg, wna_ref[...]) + bn_ref[...]
    so_ref[...] = s + _silu(h)


def _node_update(s, p0, p1, wns, wna, bn):
    return pl.pallas_call(
        _node_upd_body,
        out_shape=jax.ShapeDtypeStruct((_N, _IND), F32),
    )(s, p0, p1, wns, wna, bn.reshape(1, -1))


def _head_node_body(s_ref, wsh_ref, bsh_ref, wal_ref, bal_ref,
                    s2_ref, lat_ref, at_ref):
    s2 = _silu(_dot(s_ref[...], wsh_ref[...]) + bsh_ref[...])
    ao = _dot(s2, wal_ref[...]) + bal_ref[...]
    s2_ref[...] = s2
    at_ref[...] = ao[:, :_NAF]
    lat_ref[...] = ao[:, _NAF:]


def _head_node(s, p):
    return pl.pallas_call(
        _head_node_body,
        out_shape=(jax.ShapeDtypeStruct((_N, _IND), F32),
                   jax.ShapeDtypeStruct((_N, _LAT), F32),
                   jax.ShapeDtypeStruct((_N, _NAF), F32)),
    )(s, p['h_sh_W'], p['h_sh_b'].reshape(1, -1),
      p['h_al_W'], p['h_al_b'].reshape(1, -1))


def _head_final_body(g_ref, es_ref, wbm_ref, bbm_ref, wbl_ref, bbl_ref,
                     bo_ref):
    f = _silu(g_ref[...] + _dot(es_ref[...], wbm_ref[...]) + bbm_ref[...])
    bo_ref[...] = _dot(f, wbl_ref[...]) + bbl_ref[...]


def _head_final(gsum, esym, p):
    nbt = p['h_bl_W'].shape[1]
    grid = (_E // ET,)
    return pl.pallas_call(
        _head_final_body,
        grid=grid,
        in_specs=[
            pl.BlockSpec((ET, _IND), lambda b: (b, 0)),
            pl.BlockSpec((ET, _EDD), lambda b: (b, 0)),
            pl.BlockSpec((_EDD, _IND), lambda b: (0, 0)),
            pl.BlockSpec((1, _IND), lambda b: (0, 0)),
            pl.BlockSpec((_IND, nbt), lambda b: (0, 0)),
            pl.BlockSpec((1, nbt), lambda b: (0, 0)),
        ],
        out_specs=pl.BlockSpec((ET, nbt), lambda b: (b, 0)),
        out_shape=jax.ShapeDtypeStruct((_E, nbt), F32),
    )(gsum, esym, p['h_bm_W'], p['h_bm_b'].reshape(1, -1),
      p['h_bl_W'], p['h_bl_b'].reshape(1, -1))


def kernel(x, t, z, edge_attr, params, edge_index, batch, batch_edge_global):
    p = params
    j = edge_index[0]
    i = edge_index[1]
    batch_col = batch.reshape(_N, 1)
    beg_col = batch_edge_global.reshape(_E, 1)
    jr3 = j.reshape(_NW, _MNCH, _MC)
    ir3 = i.reshape(_NW, _MNCH, _MC)
    jr3m = j.reshape(_NW, _MNCHM, _MCM)
    ir3m = i.reshape(_NW, _MNCHM, _MCM)

    s = _node_init(x, z, batch_col, t, p)
    e = _edge_init(edge_attr, beg_col, t, p)

    epad = None
    for l in range(_LGNN):
        wmsg = p['gnn_Wmsg'][l]
        wedge = p['gnn_Wedge'][l]
        wnode = p['gnn_Wnode'][l]
        pj, pi = _node_tables(s, wmsg[:_IND], wmsg[_IND:2 * _IND])
        q, r = _edge_pre(e, wmsg[2 * _IND:], p['gnn_bmsg'][l],
                         wedge[2 * _IND:], p['gnn_bedge'][l])
        parts = _sc_msg(pj, pi, q, jr3m, ir3m)
        s = _node_update(s, parts[0], parts[1],
                         wnode[:_IND], wnode[_IND:], p['gnn_bnode'][l])
        qt = _combo_table(s, wedge[:_IND], wedge[_IND:2 * _IND])
        if l < _LGNN - 1:
            e = _sc_edge(qt, e, r, jr3, ir3)
        else:
            epad = _sc_edge_pad(qt, e, r, jr3, ir3)

    s2, latent_pred, atoms_pred = _head_node(s, p)
    pmap, gsum = _sc_head1(s2, jr3, ir3)
    esym = _sc_head2(pmap, epad, jr3, ir3)
    bonds_pred = _head_final(gsum, esym, p)
    return latent_pred, atoms_pred, bonds_pred
